# baseline (speedup 1.0000x reference)
import jax
import jax.numpy as jnp
from jax import lax
from jax.experimental import pallas as pl
from jax.experimental.pallas import tpu as pltpu
from jax.experimental.pallas import tpu_sc as plsc

B = 4
E = 80000
M = B * E
C = 128
W32 = 128

NW = 32
PER_W = M // NW
KCH = 40
NCHUNK = PER_W // KCH

BLK = 800
NEB = E // BLK


NGRP = 5
GCH = NCHUNK // NGRP


def _sc_gather_body(table, idxw, o1, o2, o3, o4, ivall,
                    b00, b01, b02, b03, b10, b11, b12, b13,
                    sg0, sg1, sw0, sw1):
    wid = lax.axis_index("s") * 2 + lax.axis_index("c")
    base0 = wid * PER_W

    bufs = ((b00, b01, b02, b03), (b10, b11, b12, b13))
    sgs = (sg0, sg1)
    sws = (sw0, sw1)
    ohs = (o1, o2, o3, o4)

    def group(g, carry):
        pltpu.sync_copy(idxw.at[wid, g], ivall)

        def fire_gathers(lci, s):
            return [pltpu.async_copy(table.at[ivall.at[lci * 4 + j]],
                                     bufs[s][j], sgs[s]) for j in range(4)]

        def fire_writes(lci, s):
            base = base0 + (g * GCH + lci) * KCH
            for j in range(4):
                pltpu.async_copy(bufs[s][j], ohs[j].at[pl.ds(base, KCH)],
                                 sws[s])

        def wait_writes(s):
            for j in range(4):
                pltpu.make_async_copy(bufs[s][j], ohs[j].at[pl.ds(0, KCH)],
                                      sws[s]).wait()

        def run_chunk(lci, s):
            cps = fire_gathers(lci, s)
            for cp in cps:
                cp.wait()
            fire_writes(lci, s)

        run_chunk(0, 0)
        run_chunk(1, 1)

        def pair(k, c2):
            for s in (0, 1):
                wait_writes(s)
                run_chunk(2 + 2 * k + s, s)
            return c2

        lax.fori_loop(0, (GCH - 2) // 2, pair, 0)
        wait_writes(0)
        wait_writes(1)
        return carry

    lax.fori_loop(0, NGRP, group, 0)


def _make_sc_gather():
    mesh = plsc.VectorSubcoreMesh(core_axis_name="c", subcore_axis_name="s")
    out_t = [jax.ShapeDtypeStruct((M, W32), jnp.int32)] * 4
    scratch = ([pltpu.VMEM((GCH * 4, KCH), jnp.int32)]
               + [pltpu.VMEM((KCH, W32), jnp.int32)] * 8
               + [pltpu.SemaphoreType.DMA] * 4)
    return pl.kernel(_sc_gather_body, mesh=mesh, out_type=out_t,
                     scratch_types=scratch)


def _f32_as_i32(x):
    return jax.lax.bitcast_convert_type(x, jnp.int32)


def _i32_as_f32(x):
    return jax.lax.bitcast_convert_type(x, jnp.float32)


def _bf16_as_i32(x):
    return jax.lax.bitcast_convert_type(x.reshape(M, W32, 2), jnp.int32)


def _i32_as_bf16(x):
    return jax.lax.bitcast_convert_type(x, jnp.bfloat16).reshape(M, 2 * W32)


def _combo(a1, a2, a3, a4):
    s13 = a1 + a3
    s24 = a2 + a4
    d13 = jnp.abs(a1 - a3)
    d24 = jnp.abs(a2 - a4)
    return [s13, s24, d13, d24]


def _conv1_body(f0, a1, a2, a3, a4, w, bias, y):
    g = jnp.concatenate(
        [f0[...]] + _combo(a1[...], a2[...], a3[...], a4[...]), axis=1)
    y[...] = jnp.dot(g, w[...], preferred_element_type=jnp.float32) + bias[...]


def _stats_epilogue(i, y, acc1, acc2, scale, shift):
    @pl.when(i == 0)
    def _():
        acc1[...] = jnp.zeros_like(acc1)
        acc2[...] = jnp.zeros_like(acc2)

    acc1[...] += jnp.sum(y, axis=0, keepdims=True)
    acc2[...] += jnp.sum(y * y, axis=0, keepdims=True)

    @pl.when(i == NEB - 1)
    def _():
        mean = acc1[...] * (1.0 / E)
        var = acc2[...] * (1.0 / E) - mean * mean
        rstd = lax.rsqrt(var + 1e-5)
        scale[...] = rstd.reshape(1, 1, C)
        shift[...] = (-mean * rstd).reshape(1, 1, C)


def _conv2_body(y1r, fdr, q1, q2, q3, q4, h1, h2, h3, h4, w, bias,
                y2, scale, shift, acc1, acc2):
    i = pl.program_id(1)
    qc = [c.astype(jnp.bfloat16)
          for c in _combo(q1[...], q2[...], q3[...], q4[...])]
    hc = _combo(h1[...], h2[...], h3[...], h4[...])
    g = jnp.concatenate([y1r[...].astype(jnp.bfloat16), fdr[...]] + qc + hc,
                        axis=1)
    y = jnp.dot(g, w[...], preferred_element_type=jnp.float32) + bias[...]
    y2[...] = y.astype(jnp.bfloat16)
    _stats_epilogue(i, y, acc1, acc2, scale, shift)


def _norm_body(y2r, scale, shift, x1):
    x1[...] = jnp.maximum(y2r[...] * scale[...].reshape(1, C)
                          + shift[...].reshape(1, C), 0.0)


def _conv3_body(x1r, a1, a2, a3, a4, w, bias, y3, scale, shift, acc1, acc2):
    i = pl.program_id(1)
    ac = [c.astype(jnp.bfloat16)
          for c in _combo(a1[...], a2[...], a3[...], a4[...])]
    g = jnp.concatenate([x1r[...].astype(jnp.bfloat16)] + ac, axis=1)
    y = jnp.dot(g, w[...], preferred_element_type=jnp.float32) + bias[...]
    y3[...] = y.astype(jnp.bfloat16)
    _stats_epilogue(i, y, acc1, acc2, scale, shift)


def _row1(col):
    return pl.BlockSpec((BLK, C), lambda i, c=col: (i, c))


def _row2(col):
    return pl.BlockSpec((BLK, C), lambda b, i, c=col: (b * NEB + i, c))


def _stat_spec():
    return pl.BlockSpec((1, 1, C), lambda b, i: (b, 0, 0))


_STAT_SHAPE = jax.ShapeDtypeStruct((B, 1, C), jnp.float32)


def _conv1_call(t1, p, wc, bias):
    return pl.pallas_call(
        _conv1_body,
        grid=(M // BLK,),
        in_specs=[_row1(0)] * 5 + [pl.BlockSpec((5 * C, C), lambda i: (0, 0)),
                                   pl.BlockSpec((1, C), lambda i: (0, 0))],
        out_specs=pl.BlockSpec((BLK, C), lambda i: (i, 0)),
        out_shape=jax.ShapeDtypeStruct((M, C), jnp.float32),
    )(t1, *p, wc, bias)


def _conv2_call(y1, t1, q, p, wc, bias):
    return pl.pallas_call(
        _conv2_body,
        grid=(B, NEB),
        in_specs=([_row2(0), _row2(1)] + [_row2(0)] * 4 + [_row2(1)] * 4
                  + [pl.BlockSpec((10 * C, C), lambda b, i: (0, 0)),
                     pl.BlockSpec((1, C), lambda b, i: (0, 0))]),
        out_specs=[_row2(0), _stat_spec(), _stat_spec()],
        out_shape=[jax.ShapeDtypeStruct((M, C), jnp.bfloat16),
                   _STAT_SHAPE, _STAT_SHAPE],
        scratch_shapes=[pltpu.VMEM((1, C), jnp.float32),
                        pltpu.VMEM((1, C), jnp.float32)],
    )(y1, t1, *q, *p, wc, bias)


def _norm_call(y2, scale, shift):
    return pl.pallas_call(
        _norm_body,
        grid=(B, NEB),
        in_specs=[_row2(0), _stat_spec(), _stat_spec()],
        out_specs=_row2(0),
        out_shape=jax.ShapeDtypeStruct((M, C), jnp.float32),
    )(y2, scale, shift)


def _conv3_call(x1, a, wc, bias):
    return pl.pallas_call(
        _conv3_body,
        grid=(B, NEB),
        in_specs=([_row2(0)] * 5
                  + [pl.BlockSpec((5 * C, C), lambda b, i: (0, 0)),
                     pl.BlockSpec((1, C), lambda b, i: (0, 0))]),
        out_specs=[_row2(0), _stat_spec(), _stat_spec()],
        out_shape=[jax.ShapeDtypeStruct((M, C), jnp.bfloat16),
                   _STAT_SHAPE, _STAT_SHAPE],
        scratch_shapes=[pltpu.VMEM((1, C), jnp.float32),
                        pltpu.VMEM((1, C), jnp.float32)],
    )(x1, *a, wc, bias)


def _final_body(y3r, x1r, scale, shift, out):
    out[...] = jnp.maximum(y3r[...] * scale[...].reshape(1, C)
                           + shift[...].reshape(1, C) + x1r[...], 0.0)


def _final_call(y3, x1, scale, shift):
    return pl.pallas_call(
        _final_body,
        grid=(B, NEB),
        in_specs=[_row2(0), _row2(0), _stat_spec(), _stat_spec()],
        out_specs=_row2(0),
        out_shape=jax.ShapeDtypeStruct((M, C), jnp.float32),
    )(y3, x1, scale, shift)


def kernel(from_up, from_down, gemm_edges, W_up, b_up, W1, b1, W2, b2):
    fu = from_up.astype(jnp.bfloat16).transpose(0, 2, 1).reshape(M, C)
    fd = from_down.astype(jnp.bfloat16).transpose(0, 2, 1).reshape(M, C)
    t1 = jnp.concatenate([fu, fd], axis=1)

    ge = (gemm_edges.astype(jnp.int32)
          + (jnp.arange(B, dtype=jnp.int32) * E)[:, None, None])
    idxw = (ge.reshape(M, 4).T
            .reshape(4, NW, NGRP, GCH, KCH)
            .transpose(1, 2, 3, 0, 4)
            .reshape(NW, NGRP, GCH * 4, KCH))

    def wcat(W, cols):
        return jnp.concatenate([W[:, cs, k].T for (cs, k) in cols],
                               axis=0).astype(jnp.bfloat16)

    full = slice(0, C)
    lo, hi = slice(0, C), slice(C, 2 * C)
    wc1 = wcat(W_up, [(full, 0), (full, 1), (full, 2), (full, 3), (full, 4)])
    wc2 = wcat(W1, [(lo, 0), (hi, 0), (lo, 1), (lo, 2), (lo, 3), (lo, 4),
                    (hi, 1), (hi, 2), (hi, 3), (hi, 4)])
    wc3 = wcat(W2, [(full, 0), (full, 1), (full, 2), (full, 3), (full, 4)])

    sc = _make_sc_gather()

    p = [_i32_as_bf16(t) for t in sc(_bf16_as_i32(t1), idxw)]
    y1 = _conv1_call(t1, p, wc1, b_up.reshape(1, C))
    q = [_i32_as_f32(t) for t in sc(_f32_as_i32(y1), idxw)]
    y2, scale2, shift2 = _conv2_call(y1, t1, q, p, wc2, b1.reshape(1, C))
    x1 = _norm_call(y2, scale2, shift2)
    a = [_i32_as_f32(t) for t in sc(_f32_as_i32(x1), idxw)]
    y3, scale3, shift3 = _conv3_call(x1, a, wc3, b2.reshape(1, C))
    out = _final_call(y3, x1, scale3, shift3)
    return out.reshape(B, E, C).transpose(0, 2, 1)


# speedup vs baseline: 2.1396x; 2.1396x over previous
import jax
import jax.numpy as jnp
import numpy as np
from jax import lax
from jax.experimental import pallas as pl
from jax.experimental.pallas import tpu as pltpu
from jax.experimental.pallas import tpu_sc as plsc

B = 4
E = 80000
M = B * E
C = 128

NW = 32
PER_W = M // NW
KCH = 40
NCHUNK = PER_W // KCH
NGRP = 5
GCH = NCHUNK // NGRP

BLK = 800
NEB = E // BLK


def _sc_gather_body(table, idxw, o1, o2, o3, o4, ivall,
                    b00, b01, b02, b03, b10, b11, b12, b13,
                    sg0, sg1, sw0, sw1):
    wid = lax.axis_index("s") * 2 + lax.axis_index("c")
    base0 = wid * PER_W

    bufs = ((b00, b01, b02, b03), (b10, b11, b12, b13))
    sgs = (sg0, sg1)
    sws = (sw0, sw1)
    ohs = (o1, o2, o3, o4)

    def group(g, carry):
        pltpu.sync_copy(idxw.at[wid, g], ivall)

        def fire_gathers(lci, s):
            return [pltpu.async_copy(table.at[ivall.at[lci * 4 + j]],
                                     bufs[s][j], sgs[s]) for j in range(4)]

        def fire_writes(lci, s):
            base = base0 + (g * GCH + lci) * KCH
            for j in range(4):
                pltpu.async_copy(bufs[s][j], ohs[j].at[pl.ds(base, KCH)],
                                 sws[s])

        def wait_writes(s):
            for j in range(4):
                pltpu.make_async_copy(bufs[s][j], ohs[j].at[pl.ds(0, KCH)],
                                      sws[s]).wait()

        def run_chunk(lci, s):
            cps = fire_gathers(lci, s)
            for cp in cps:
                cp.wait()
            fire_writes(lci, s)

        run_chunk(0, 0)
        run_chunk(1, 1)

        def pair(k, c2):
            for s in (0, 1):
                wait_writes(s)
                run_chunk(2 + 2 * k + s, s)
            return c2

        lax.fori_loop(0, (GCH - 2) // 2, pair, 0)
        wait_writes(0)
        wait_writes(1)
        return carry

    lax.fori_loop(0, NGRP, group, 0)


def _make_sc_gather():
    mesh = plsc.VectorSubcoreMesh(core_axis_name="c", subcore_axis_name="s")
    out_t = [jax.ShapeDtypeStruct((M, C), jnp.int32)] * 4
    scratch = ([pltpu.VMEM((GCH * 4, KCH), jnp.int32)]
               + [pltpu.VMEM((KCH, C), jnp.int32)] * 8
               + [pltpu.SemaphoreType.DMA] * 4)
    return pl.kernel(_sc_gather_body, mesh=mesh, out_type=out_t,
                     scratch_types=scratch)


def _f32_as_i32(x):
    return jax.lax.bitcast_convert_type(x, jnp.int32)


def _i32_as_f32(x):
    return jax.lax.bitcast_convert_type(x, jnp.float32)


_HI = np.uint32(0xFFFF0000)


def _pack2(a, b):
    ah = lax.bitcast_convert_type(a.astype(jnp.bfloat16).astype(jnp.float32),
                                  jnp.uint32)
    bh = lax.bitcast_convert_type(b.astype(jnp.bfloat16).astype(jnp.float32),
                                  jnp.uint32)
    return lax.bitcast_convert_type((ah & _HI) | (bh >> 16), jnp.int32)


def _unpack_hi(p):
    u = lax.bitcast_convert_type(p, jnp.uint32)
    return lax.bitcast_convert_type(u & _HI, jnp.float32)


def _unpack_lo(p):
    u = lax.bitcast_convert_type(p, jnp.uint32)
    return lax.bitcast_convert_type(u << 16, jnp.float32)


def _combo(a1, a2, a3, a4):
    return [a1 + a3, a2 + a4, jnp.abs(a1 - a3), jnp.abs(a2 - a4)]


def _bf(xs):
    return [x.astype(jnp.bfloat16) for x in xs]


def _pack_body(fu, fd, t1):
    t1[...] = _pack2(fu[...], fd[...])


def _conv1_body(fu, fd, p1, p2, p3, p4, w1, wz, b1, bz, y1, z):
    hi = [_unpack_hi(p[...]) for p in (p1, p2, p3, p4)]
    lo = [_unpack_lo(p[...]) for p in (p1, p2, p3, p4)]
    gy = jnp.concatenate([fu[...].astype(jnp.bfloat16)] + _bf(_combo(*hi)),
                         axis=1)
    gz = jnp.concatenate([fd[...].astype(jnp.bfloat16)] + _bf(_combo(*lo)),
                         axis=1)
    y1[...] = jnp.dot(gy, w1[...], preferred_element_type=jnp.float32) + b1[...]
    z[...] = jnp.dot(gz, wz[...], preferred_element_type=jnp.float32) + bz[...]


def _stats_epilogue(i, y, acc1, acc2, scale, shift):
    @pl.when(i == 0)
    def _():
        acc1[...] = jnp.zeros_like(acc1)
        acc2[...] = jnp.zeros_like(acc2)

    acc1[...] += jnp.sum(y, axis=0, keepdims=True)
    acc2[...] += jnp.sum(y * y, axis=0, keepdims=True)

    @pl.when(i == NEB - 1)
    def _():
        mean = acc1[...] * (1.0 / E)
        var = acc2[...] * (1.0 / E) - mean * mean
        rstd = lax.rsqrt(var + 1e-5)
        scale[...] = rstd.reshape(1, 1, C)
        shift[...] = (-mean * rstd).reshape(1, 1, C)


def _conv2_body(y1r, q1, q2, q3, q4, z, w, y2, scale, shift, acc1, acc2):
    i = pl.program_id(1)
    g = jnp.concatenate([y1r[...].astype(jnp.bfloat16)]
                        + _bf(_combo(q1[...], q2[...], q3[...], q4[...])),
                        axis=1)
    y = jnp.dot(g, w[...], preferred_element_type=jnp.float32) + z[...]
    y2[...] = y.astype(jnp.bfloat16)
    _stats_epilogue(i, y, acc1, acc2, scale, shift)


def _norm_body(y2r, scale, shift, x1):
    x1[...] = jnp.maximum(y2r[...] * scale[...].reshape(1, C)
                          + shift[...].reshape(1, C), 0.0)


def _conv3_body(x1r, a1, a2, a3, a4, w, bias, y3, scale, shift, acc1, acc2):
    i = pl.program_id(1)
    g = jnp.concatenate([x1r[...].astype(jnp.bfloat16)]
                        + _bf(_combo(a1[...], a2[...], a3[...], a4[...])),
                        axis=1)
    y = jnp.dot(g, w[...], preferred_element_type=jnp.float32) + bias[...]
    y3[...] = y.astype(jnp.bfloat16)
    _stats_epilogue(i, y, acc1, acc2, scale, shift)


def _final_body(y3r, x1r, scale, shift, out):
    out[...] = jnp.maximum(y3r[...] * scale[...].reshape(1, C)
                           + shift[...].reshape(1, C) + x1r[...], 0.0)


def _row1():
    return pl.BlockSpec((BLK, C), lambda i: (i, 0))


def _row2():
    return pl.BlockSpec((BLK, C), lambda b, i: (b * NEB + i, 0))


def _w1_spec(k):
    return pl.BlockSpec((k, C), lambda i: (0, 0))


def _w2_spec(k):
    return pl.BlockSpec((k, C), lambda b, i: (0, 0))


def _stat_spec():
    return pl.BlockSpec((1, 1, C), lambda b, i: (b, 0, 0))


_STAT_SHAPE = jax.ShapeDtypeStruct((B, 1, C), jnp.float32)
_ROW_F32 = jax.ShapeDtypeStruct((M, C), jnp.float32)
_ROW_BF16 = jax.ShapeDtypeStruct((M, C), jnp.bfloat16)


def _pack_call(fu, fd):
    return pl.pallas_call(
        _pack_body, grid=(M // BLK,),
        in_specs=[_row1(), _row1()], out_specs=_row1(),
        out_shape=jax.ShapeDtypeStruct((M, C), jnp.int32),
    )(fu, fd)


def _conv1_call(fu, fd, p, w1, wz, b1, bz):
    return pl.pallas_call(
        _conv1_body, grid=(M // BLK,),
        in_specs=[_row1()] * 6 + [_w1_spec(5 * C), _w1_spec(5 * C),
                                  _w1_spec(1), _w1_spec(1)],
        out_specs=[_row1(), _row1()],
        out_shape=[_ROW_F32, _ROW_F32],
    )(fu, fd, *p, w1, wz, b1, bz)


def _conv2_call(y1, q, z, wc):
    return pl.pallas_call(
        _conv2_body, grid=(B, NEB),
        in_specs=[_row2()] * 6 + [_w2_spec(5 * C)],
        out_specs=[_row2(), _stat_spec(), _stat_spec()],
        out_shape=[_ROW_BF16, _STAT_SHAPE, _STAT_SHAPE],
        scratch_shapes=[pltpu.VMEM((1, C), jnp.float32),
                        pltpu.VMEM((1, C), jnp.float32)],
    )(y1, *q, z, wc)


def _norm_call(y2, scale, shift):
    return pl.pallas_call(
        _norm_body, grid=(B, NEB),
        in_specs=[_row2(), _stat_spec(), _stat_spec()],
        out_specs=_row2(),
        out_shape=_ROW_F32,
    )(y2, scale, shift)


def _conv3_call(x1, a, wc, bias):
    return pl.pallas_call(
        _conv3_body, grid=(B, NEB),
        in_specs=[_row2()] * 5 + [_w2_spec(5 * C), _w2_spec(1)],
        out_specs=[_row2(), _stat_spec(), _stat_spec()],
        out_shape=[_ROW_BF16, _STAT_SHAPE, _STAT_SHAPE],
        scratch_shapes=[pltpu.VMEM((1, C), jnp.float32),
                        pltpu.VMEM((1, C), jnp.float32)],
    )(x1, *a, wc, bias)


def _final_call(y3, x1, scale, shift):
    return pl.pallas_call(
        _final_body, grid=(B, NEB),
        in_specs=[_row2(), _row2(), _stat_spec(), _stat_spec()],
        out_specs=_row2(),
        out_shape=_ROW_F32,
    )(y3, x1, scale, shift)


def kernel(from_up, from_down, gemm_edges, W_up, b_up, W1, b1, W2, b2):
    fu = from_up.transpose(0, 2, 1).reshape(M, C)
    fd = from_down.transpose(0, 2, 1).reshape(M, C)

    ge = (gemm_edges.astype(jnp.int32)
          + (jnp.arange(B, dtype=jnp.int32) * E)[:, None, None])
    idxw = (ge.reshape(M, 4).T
            .reshape(4, NW, NGRP, GCH, KCH)
            .transpose(1, 2, 3, 0, 4)
            .reshape(NW, NGRP, GCH * 4, KCH))

    def wcat(W, cols):
        return jnp.concatenate([W[:, cs, k].T for (cs, k) in cols],
                               axis=0).astype(jnp.bfloat16)

    full = slice(0, C)
    lo, hi = slice(0, C), slice(C, 2 * C)
    taps5 = [0, 1, 2, 3, 4]
    wc1 = wcat(W_up, [(full, k) for k in taps5])
    wcz = wcat(W1, [(hi, k) for k in taps5])
    wc2 = wcat(W1, [(lo, k) for k in taps5])
    wc3 = wcat(W2, [(full, k) for k in taps5])

    sc = _make_sc_gather()

    t1 = _pack_call(fu, fd)
    p = sc(t1, idxw)
    y1, z = _conv1_call(fu, fd, p, wc1, wcz,
                        b_up.reshape(1, C), b1.reshape(1, C))
    q = [_i32_as_f32(t) for t in sc(_f32_as_i32(y1), idxw)]
    y2, scale2, shift2 = _conv2_call(y1, q, z, wc2)
    x1 = _norm_call(y2, scale2, shift2)
    a = [_i32_as_f32(t) for t in sc(_f32_as_i32(x1), idxw)]
    y3, scale3, shift3 = _conv3_call(x1, a, wc3, b2.reshape(1, C))
    out = _final_call(y3, x1, scale3, shift3)
    return out.reshape(B, E, C).transpose(0, 2, 1)


# speedup vs baseline: 3.1910x; 1.4914x over previous
import jax
import jax.numpy as jnp
import numpy as np
from jax import lax
from jax.experimental import pallas as pl
from jax.experimental.pallas import tpu as pltpu
from jax.experimental.pallas import tpu_sc as plsc

B = 4
E = 80000
M = B * E
C = 128

NW = 32
PER_W = M // NW
KCH = 40
NCHUNK = PER_W // KCH
NGRP = 5
GCH = NCHUNK // NGRP

BLK = 2000
NEB = E // BLK


def _sc_gather_body(table, idxw, o1, o2, o3, o4, ivall,
                    b00, b01, b02, b03, b10, b11, b12, b13,
                    sg0, sg1, sw0, sw1):
    wid = lax.axis_index("s") * 2 + lax.axis_index("c")
    base0 = wid * PER_W

    bufs = ((b00, b01, b02, b03), (b10, b11, b12, b13))
    sgs = (sg0, sg1)
    sws = (sw0, sw1)
    ohs = (o1, o2, o3, o4)

    def group(g, carry):
        pltpu.sync_copy(idxw.at[wid, g], ivall)

        def fire_gathers(lci, s):
            return [pltpu.async_copy(table.at[ivall.at[lci * 4 + j]],
                                     bufs[s][j], sgs[s]) for j in range(4)]

        def fire_writes(lci, s):
            base = base0 + (g * GCH + lci) * KCH
            for j in range(4):
                pltpu.async_copy(bufs[s][j], ohs[j].at[pl.ds(base, KCH)],
                                 sws[s])

        def wait_writes(s):
            for j in range(4):
                pltpu.make_async_copy(bufs[s][j], ohs[j].at[pl.ds(0, KCH)],
                                      sws[s]).wait()

        def run_chunk(lci, s):
            cps = fire_gathers(lci, s)
            for cp in cps:
                cp.wait()
            fire_writes(lci, s)

        run_chunk(0, 0)
        run_chunk(1, 1)

        def pair(k, c2):
            for s in (0, 1):
                wait_writes(s)
                run_chunk(2 + 2 * k + s, s)
            return c2

        lax.fori_loop(0, (GCH - 2) // 2, pair, 0)
        wait_writes(0)
        wait_writes(1)
        return carry

    lax.fori_loop(0, NGRP, group, 0)


def _make_sc_gather(dtype):
    mesh = plsc.VectorSubcoreMesh(core_axis_name="c", subcore_axis_name="s")
    out_t = [jax.ShapeDtypeStruct((M, C), dtype)] * 4
    scratch = ([pltpu.VMEM((GCH * 4, KCH), jnp.int32)]
               + [pltpu.VMEM((KCH, C), dtype)] * 8
               + [pltpu.SemaphoreType.DMA] * 4)
    return pl.kernel(_sc_gather_body, mesh=mesh, out_type=out_t,
                     scratch_types=scratch)


_HI = np.uint32(0xFFFF0000)


def _pack2(a, b):
    ah = lax.bitcast_convert_type(a.astype(jnp.bfloat16).astype(jnp.float32),
                                  jnp.uint32)
    bh = lax.bitcast_convert_type(b.astype(jnp.bfloat16).astype(jnp.float32),
                                  jnp.uint32)
    return lax.bitcast_convert_type((ah & _HI) | (bh >> 16), jnp.int32)


def _unpack_hi(p):
    u = lax.bitcast_convert_type(p, jnp.uint32)
    return lax.bitcast_convert_type(u & _HI, jnp.float32)


def _unpack_lo(p):
    u = lax.bitcast_convert_type(p, jnp.uint32)
    return lax.bitcast_convert_type(u << 16, jnp.float32)


def _combo(a1, a2, a3, a4):
    return [a1 + a3, a2 + a4, jnp.abs(a1 - a3), jnp.abs(a2 - a4)]


def _bf(xs):
    return [x.astype(jnp.bfloat16) for x in xs]


def _pack_body(fu, fd, t1):
    t1[...] = _pack2(fu[...], fd[...])


def _conv1_body(fu, fd, p1, p2, p3, p4, w1, wz, b1, bz, y1, z):
    hi = [_unpack_hi(p[...]) for p in (p1, p2, p3, p4)]
    lo = [_unpack_lo(p[...]) for p in (p1, p2, p3, p4)]
    gy = jnp.concatenate([fu[...].astype(jnp.bfloat16)] + _bf(_combo(*hi)),
                         axis=1)
    gz = jnp.concatenate([fd[...].astype(jnp.bfloat16)] + _bf(_combo(*lo)),
                         axis=1)
    y1[...] = jnp.dot(gy, w1[...], preferred_element_type=jnp.float32) + b1[...]
    z[...] = jnp.dot(gz, wz[...], preferred_element_type=jnp.float32) + bz[...]


def _stats_epilogue(i, y, acc1, acc2, scale, shift):
    @pl.when(i == 0)
    def _():
        acc1[...] = jnp.zeros_like(acc1)
        acc2[...] = jnp.zeros_like(acc2)

    acc1[...] += jnp.sum(y, axis=0, keepdims=True)
    acc2[...] += jnp.sum(y * y, axis=0, keepdims=True)

    @pl.when(i == NEB - 1)
    def _():
        mean = acc1[...] * (1.0 / E)
        var = acc2[...] * (1.0 / E) - mean * mean
        rstd = lax.rsqrt(var + 1e-5)
        scale[...] = rstd.reshape(1, 1, C)
        shift[...] = (-mean * rstd).reshape(1, 1, C)


def _conv2_body(y1r, q1, q2, q3, q4, z, w, y2, scale, shift, acc1, acc2):
    i = pl.program_id(1)
    g = jnp.concatenate([y1r[...].astype(jnp.bfloat16)]
                        + _bf(_combo(q1[...], q2[...], q3[...], q4[...])),
                        axis=1)
    y = jnp.dot(g, w[...], preferred_element_type=jnp.float32) + z[...]
    y2[...] = y.astype(jnp.bfloat16)
    _stats_epilogue(i, y, acc1, acc2, scale, shift)


def _norm_body(y2r, scale, shift, x1):
    x1[...] = jnp.maximum(y2r[...] * scale[...].reshape(1, C)
                          + shift[...].reshape(1, C), 0.0)


def _conv3_body(x1r, a1, a2, a3, a4, w, bias, y3, scale, shift, acc1, acc2):
    i = pl.program_id(1)
    g = jnp.concatenate([x1r[...].astype(jnp.bfloat16)]
                        + _bf(_combo(a1[...], a2[...], a3[...], a4[...])),
                        axis=1)
    y = jnp.dot(g, w[...], preferred_element_type=jnp.float32) + bias[...]
    y3[...] = y.astype(jnp.bfloat16)
    _stats_epilogue(i, y, acc1, acc2, scale, shift)


def _final_body(y3r, x1r, scale, shift, out):
    out[...] = jnp.maximum(y3r[...] * scale[...].reshape(1, C)
                           + shift[...].reshape(1, C) + x1r[...], 0.0)


def _row1():
    return pl.BlockSpec((BLK, C), lambda i: (i, 0))


def _row2():
    return pl.BlockSpec((BLK, C), lambda b, i: (b * NEB + i, 0))


def _w1_spec(k):
    return pl.BlockSpec((k, C), lambda i: (0, 0))


def _w2_spec(k):
    return pl.BlockSpec((k, C), lambda b, i: (0, 0))


def _stat_spec():
    return pl.BlockSpec((1, 1, C), lambda b, i: (b, 0, 0))


_STAT_SHAPE = jax.ShapeDtypeStruct((B, 1, C), jnp.float32)
_ROW_F32 = jax.ShapeDtypeStruct((M, C), jnp.float32)
_ROW_BF16 = jax.ShapeDtypeStruct((M, C), jnp.bfloat16)


def _pack_call(fu, fd):
    return pl.pallas_call(
        _pack_body, grid=(M // BLK,),
        in_specs=[_row1(), _row1()], out_specs=_row1(),
        out_shape=jax.ShapeDtypeStruct((M, C), jnp.int32),
    )(fu, fd)


def _conv1_call(fu, fd, p, w1, wz, b1, bz):
    return pl.pallas_call(
        _conv1_body, grid=(M // BLK,),
        in_specs=[_row1()] * 6 + [_w1_spec(5 * C), _w1_spec(5 * C),
                                  _w1_spec(1), _w1_spec(1)],
        out_specs=[_row1(), _row1()],
        out_shape=[_ROW_F32, _ROW_F32],
    )(fu, fd, *p, w1, wz, b1, bz)


def _conv2_call(y1, q, z, wc):
    return pl.pallas_call(
        _conv2_body, grid=(B, NEB),
        in_specs=[_row2()] * 6 + [_w2_spec(5 * C)],
        out_specs=[_row2(), _stat_spec(), _stat_spec()],
        out_shape=[_ROW_BF16, _STAT_SHAPE, _STAT_SHAPE],
        scratch_shapes=[pltpu.VMEM((1, C), jnp.float32),
                        pltpu.VMEM((1, C), jnp.float32)],
    )(y1, *q, z, wc)


def _norm_call(y2, scale, shift):
    return pl.pallas_call(
        _norm_body, grid=(B, NEB),
        in_specs=[_row2(), _stat_spec(), _stat_spec()],
        out_specs=_row2(),
        out_shape=_ROW_F32,
    )(y2, scale, shift)


def _conv3_call(x1, a, wc, bias):
    return pl.pallas_call(
        _conv3_body, grid=(B, NEB),
        in_specs=[_row2()] * 5 + [_w2_spec(5 * C), _w2_spec(1)],
        out_specs=[_row2(), _stat_spec(), _stat_spec()],
        out_shape=[_ROW_BF16, _STAT_SHAPE, _STAT_SHAPE],
        scratch_shapes=[pltpu.VMEM((1, C), jnp.float32),
                        pltpu.VMEM((1, C), jnp.float32)],
    )(x1, *a, wc, bias)


def _final_call(y3, x1, scale, shift):
    return pl.pallas_call(
        _final_body, grid=(B, NEB),
        in_specs=[_row2(), _row2(), _stat_spec(), _stat_spec()],
        out_specs=_row2(),
        out_shape=_ROW_F32,
    )(y3, x1, scale, shift)


def kernel(from_up, from_down, gemm_edges, W_up, b_up, W1, b1, W2, b2):
    fu = from_up.transpose(0, 2, 1).reshape(M, C)
    fd = from_down.transpose(0, 2, 1).reshape(M, C)

    ge = (gemm_edges.astype(jnp.int32)
          + (jnp.arange(B, dtype=jnp.int32) * E)[:, None, None])
    idxw = (ge.reshape(M, 4).T
            .reshape(4, NW, NGRP, GCH, KCH)
            .transpose(1, 2, 3, 0, 4)
            .reshape(NW, NGRP, GCH * 4, KCH))

    def wcat(W, cols):
        return jnp.concatenate([W[:, cs, k].T for (cs, k) in cols],
                               axis=0).astype(jnp.bfloat16)

    full = slice(0, C)
    lo, hi = slice(0, C), slice(C, 2 * C)
    taps5 = [0, 1, 2, 3, 4]
    wc1 = wcat(W_up, [(full, k) for k in taps5])
    wcz = wcat(W1, [(hi, k) for k in taps5])
    wc2 = wcat(W1, [(lo, k) for k in taps5])
    wc3 = wcat(W2, [(full, k) for k in taps5])

    sc_i = _make_sc_gather(jnp.int32)
    sc_f = _make_sc_gather(jnp.float32)

    t1 = _pack_call(fu, fd)
    p = sc_i(t1, idxw)
    y1, z = _conv1_call(fu, fd, p, wc1, wcz,
                        b_up.reshape(1, C), b1.reshape(1, C))
    q = sc_f(y1, idxw)
    y2, scale2, shift2 = _conv2_call(y1, q, z, wc2)
    x1 = _norm_call(y2, scale2, shift2)
    a = sc_f(x1, idxw)
    y3, scale3, shift3 = _conv3_call(x1, a, wc3, b2.reshape(1, C))
    out = _final_call(y3, x1, scale3, shift3)
    return out.reshape(B, E, C).transpose(0, 2, 1)


# speedup vs baseline: 3.2009x; 1.0031x over previous
import jax
import jax.numpy as jnp
import numpy as np
from jax import lax
from jax.experimental import pallas as pl
from jax.experimental.pallas import tpu as pltpu
from jax.experimental.pallas import tpu_sc as plsc

B = 4
E = 80000
M = B * E
C = 128

BH = 2
MH = BH * E

NW = 32
PER_W = MH // NW
KCH = 40
NCHUNK = PER_W // KCH
NGRP = 5
GCH = NCHUNK // NGRP

BLK = 2000
NEB = E // BLK


def _sc_gather_body(table, idxw, o1, o2, o3, o4, ivall,
                    b00, b01, b02, b03, b10, b11, b12, b13,
                    sg0, sg1, sw0, sw1):
    wid = lax.axis_index("s") * 2 + lax.axis_index("c")
    base0 = wid * PER_W

    bufs = ((b00, b01, b02, b03), (b10, b11, b12, b13))
    sgs = (sg0, sg1)
    sws = (sw0, sw1)
    ohs = (o1, o2, o3, o4)

    def group(g, carry):
        pltpu.sync_copy(idxw.at[wid, g], ivall)

        def fire_gathers(lci, s):
            return [pltpu.async_copy(table.at[ivall.at[lci * 4 + j]],
                                     bufs[s][j], sgs[s]) for j in range(4)]

        def fire_writes(lci, s):
            base = base0 + (g * GCH + lci) * KCH
            for j in range(4):
                pltpu.async_copy(bufs[s][j], ohs[j].at[pl.ds(base, KCH)],
                                 sws[s])

        def wait_writes(s):
            for j in range(4):
                pltpu.make_async_copy(bufs[s][j], ohs[j].at[pl.ds(0, KCH)],
                                      sws[s]).wait()

        def run_chunk(lci, s):
            cps = fire_gathers(lci, s)
            for cp in cps:
                cp.wait()
            fire_writes(lci, s)

        run_chunk(0, 0)
        run_chunk(1, 1)

        def pair(k, c2):
            for s in (0, 1):
                wait_writes(s)
                run_chunk(2 + 2 * k + s, s)
            return c2

        lax.fori_loop(0, (GCH - 2) // 2, pair, 0)
        if (GCH - 2) % 2 == 1:
            wait_writes(0)
            run_chunk(GCH - 1, 0)
        wait_writes(0)
        wait_writes(1)
        return carry

    lax.fori_loop(0, NGRP, group, 0)


def _make_sc_gather(dtype):
    mesh = plsc.VectorSubcoreMesh(core_axis_name="c", subcore_axis_name="s")
    out_t = [jax.ShapeDtypeStruct((MH, C), dtype)] * 4
    scratch = ([pltpu.VMEM((GCH * 4, KCH), jnp.int32)]
               + [pltpu.VMEM((KCH, C), dtype)] * 8
               + [pltpu.SemaphoreType.DMA] * 4)
    return pl.kernel(_sc_gather_body, mesh=mesh, out_type=out_t,
                     scratch_types=scratch)


_HI = np.uint32(0xFFFF0000)


def _pack2(a, b):
    ah = lax.bitcast_convert_type(a.astype(jnp.bfloat16).astype(jnp.float32),
                                  jnp.uint32)
    bh = lax.bitcast_convert_type(b.astype(jnp.bfloat16).astype(jnp.float32),
                                  jnp.uint32)
    return lax.bitcast_convert_type((ah & _HI) | (bh >> 16), jnp.int32)


def _unpack_hi(p):
    u = lax.bitcast_convert_type(p, jnp.uint32)
    return lax.bitcast_convert_type(u & _HI, jnp.float32)


def _unpack_lo(p):
    u = lax.bitcast_convert_type(p, jnp.uint32)
    return lax.bitcast_convert_type(u << 16, jnp.float32)


def _combo(a1, a2, a3, a4):
    return [a1 + a3, a2 + a4, jnp.abs(a1 - a3), jnp.abs(a2 - a4)]


def _bf(xs):
    return [x.astype(jnp.bfloat16) for x in xs]


def _pack_body(fu, fd, t1):
    t1[...] = _pack2(fu[...], fd[...])


def _conv1_body(fu, fd, p1, p2, p3, p4, w1, wz, b1, bz, y1, z):
    hi = [_unpack_hi(p[...]) for p in (p1, p2, p3, p4)]
    lo = [_unpack_lo(p[...]) for p in (p1, p2, p3, p4)]
    gy = jnp.concatenate([fu[...].astype(jnp.bfloat16)] + _bf(_combo(*hi)),
                         axis=1)
    gz = jnp.concatenate([fd[...].astype(jnp.bfloat16)] + _bf(_combo(*lo)),
                         axis=1)
    y1[...] = jnp.dot(gy, w1[...], preferred_element_type=jnp.float32) + b1[...]
    z[...] = jnp.dot(gz, wz[...], preferred_element_type=jnp.float32) + bz[...]


def _stats_epilogue(i, y, acc1, acc2, scale, shift):
    @pl.when(i == 0)
    def _():
        acc1[...] = jnp.zeros_like(acc1)
        acc2[...] = jnp.zeros_like(acc2)

    acc1[...] += jnp.sum(y, axis=0, keepdims=True)
    acc2[...] += jnp.sum(y * y, axis=0, keepdims=True)

    @pl.when(i == NEB - 1)
    def _():
        mean = acc1[...] * (1.0 / E)
        var = acc2[...] * (1.0 / E) - mean * mean
        rstd = lax.rsqrt(var + 1e-5)
        scale[...] = rstd.reshape(1, 1, C)
        shift[...] = (-mean * rstd).reshape(1, 1, C)


def _conv2_body(y1r, q1, q2, q3, q4, z, w, y2, scale, shift, acc1, acc2):
    i = pl.program_id(1)
    g = jnp.concatenate([y1r[...].astype(jnp.bfloat16)]
                        + _bf(_combo(q1[...], q2[...], q3[...], q4[...])),
                        axis=1)
    y = jnp.dot(g, w[...], preferred_element_type=jnp.float32) + z[...]
    y2[...] = y.astype(jnp.bfloat16)
    _stats_epilogue(i, y, acc1, acc2, scale, shift)


def _norm_body(y2r, scale, shift, x1):
    x1[...] = jnp.maximum(y2r[...] * scale[...].reshape(1, C)
                          + shift[...].reshape(1, C), 0.0)


def _conv3_body(x1r, a1, a2, a3, a4, w, bias, y3, scale, shift, acc1, acc2):
    i = pl.program_id(1)
    g = jnp.concatenate([x1r[...].astype(jnp.bfloat16)]
                        + _bf(_combo(a1[...], a2[...], a3[...], a4[...])),
                        axis=1)
    y = jnp.dot(g, w[...], preferred_element_type=jnp.float32) + bias[...]
    y3[...] = y.astype(jnp.bfloat16)
    _stats_epilogue(i, y, acc1, acc2, scale, shift)


def _final_body(y3r, x1r, scale, shift, out):
    out[...] = jnp.maximum(y3r[...] * scale[...].reshape(1, C)
                           + shift[...].reshape(1, C) + x1r[...], 0.0)


def _row1():
    return pl.BlockSpec((BLK, C), lambda i: (i, 0))


def _row2():
    return pl.BlockSpec((BLK, C), lambda b, i: (b * NEB + i, 0))


def _w1_spec(k):
    return pl.BlockSpec((k, C), lambda i: (0, 0))


def _w2_spec(k):
    return pl.BlockSpec((k, C), lambda b, i: (0, 0))


def _stat_spec():
    return pl.BlockSpec((1, 1, C), lambda b, i: (b, 0, 0))


_STAT_SHAPE = jax.ShapeDtypeStruct((BH, 1, C), jnp.float32)
_ROW_F32 = jax.ShapeDtypeStruct((MH, C), jnp.float32)
_ROW_BF16 = jax.ShapeDtypeStruct((MH, C), jnp.bfloat16)


def _pack_call(fu, fd):
    return pl.pallas_call(
        _pack_body, grid=(MH // BLK,),
        in_specs=[_row1(), _row1()], out_specs=_row1(),
        out_shape=jax.ShapeDtypeStruct((MH, C), jnp.int32),
    )(fu, fd)


def _conv1_call(fu, fd, p, w1, wz, b1, bz):
    return pl.pallas_call(
        _conv1_body, grid=(MH // BLK,),
        in_specs=[_row1()] * 6 + [_w1_spec(5 * C), _w1_spec(5 * C),
                                  _w1_spec(1), _w1_spec(1)],
        out_specs=[_row1(), _row1()],
        out_shape=[_ROW_F32, _ROW_F32],
    )(fu, fd, *p, w1, wz, b1, bz)


def _conv2_call(y1, q, z, wc):
    return pl.pallas_call(
        _conv2_body, grid=(BH, NEB),
        in_specs=[_row2()] * 6 + [_w2_spec(5 * C)],
        out_specs=[_row2(), _stat_spec(), _stat_spec()],
        out_shape=[_ROW_BF16, _STAT_SHAPE, _STAT_SHAPE],
        scratch_shapes=[pltpu.VMEM((1, C), jnp.float32),
                        pltpu.VMEM((1, C), jnp.float32)],
    )(y1, *q, z, wc)


def _norm_call(y2, scale, shift):
    return pl.pallas_call(
        _norm_body, grid=(BH, NEB),
        in_specs=[_row2(), _stat_spec(), _stat_spec()],
        out_specs=_row2(),
        out_shape=_ROW_F32,
    )(y2, scale, shift)


def _conv3_call(x1, a, wc, bias):
    return pl.pallas_call(
        _conv3_body, grid=(BH, NEB),
        in_specs=[_row2()] * 5 + [_w2_spec(5 * C), _w2_spec(1)],
        out_specs=[_row2(), _stat_spec(), _stat_spec()],
        out_shape=[_ROW_BF16, _STAT_SHAPE, _STAT_SHAPE],
        scratch_shapes=[pltpu.VMEM((1, C), jnp.float32),
                        pltpu.VMEM((1, C), jnp.float32)],
    )(x1, *a, wc, bias)


def _final_call(y3, x1, scale, shift):
    return pl.pallas_call(
        _final_body, grid=(BH, NEB),
        in_specs=[_row2(), _row2(), _stat_spec(), _stat_spec()],
        out_specs=_row2(),
        out_shape=_ROW_F32,
    )(y3, x1, scale, shift)


def kernel(from_up, from_down, gemm_edges, W_up, b_up, W1, b1, W2, b2):
    def wcat(W, cols):
        return jnp.concatenate([W[:, cs, k].T for (cs, k) in cols],
                               axis=0).astype(jnp.bfloat16)

    full = slice(0, C)
    lo, hi = slice(0, C), slice(C, 2 * C)
    taps5 = [0, 1, 2, 3, 4]
    wc1 = wcat(W_up, [(full, k) for k in taps5])
    wcz = wcat(W1, [(hi, k) for k in taps5])
    wc2 = wcat(W1, [(lo, k) for k in taps5])
    wc3 = wcat(W2, [(full, k) for k in taps5])
    bu = b_up.reshape(1, C)
    bz = b1.reshape(1, C)
    b2r = b2.reshape(1, C)

    sc_i = _make_sc_gather(jnp.int32)
    sc_f = _make_sc_gather(jnp.float32)

    H = B // BH
    fu, fd, idxw = [], [], []
    for h in range(H):
        sl = slice(h * BH, (h + 1) * BH)
        fu.append(from_up[sl].transpose(0, 2, 1).reshape(MH, C))
        fd.append(from_down[sl].transpose(0, 2, 1).reshape(MH, C))
        ge = (gemm_edges[sl].astype(jnp.int32)
              + (jnp.arange(BH, dtype=jnp.int32) * E)[:, None, None])
        idxw.append(ge.reshape(MH, 4).T
                    .reshape(4, NW, NGRP, GCH, KCH)
                    .transpose(1, 2, 3, 0, 4)
                    .reshape(NW, NGRP, GCH * 4, KCH))

    t1 = [_pack_call(fu[h], fd[h]) for h in range(H)]
    p = [sc_i(t1[h], idxw[h]) for h in range(H)]
    y1z = [_conv1_call(fu[h], fd[h], p[h], wc1, wcz, bu, bz) for h in range(H)]
    q = [sc_f(y1z[h][0], idxw[h]) for h in range(H)]
    y2s = [_conv2_call(y1z[h][0], q[h], y1z[h][1], wc2) for h in range(H)]
    x1 = [_norm_call(*y2s[h]) for h in range(H)]
    a = [sc_f(x1[h], idxw[h]) for h in range(H)]
    y3s = [_conv3_call(x1[h], a[h], wc3, b2r) for h in range(H)]
    out = [_final_call(y3s[h][0], x1[h], y3s[h][1], y3s[h][2])
           for h in range(H)]
    outc = jnp.concatenate(out, axis=0)
    return outc.reshape(B, E, C).transpose(0, 2, 1)


# speedup vs baseline: 3.6533x; 1.1413x over previous
import jax
import jax.numpy as jnp
import numpy as np
from jax import lax
from jax.experimental import pallas as pl
from jax.experimental.pallas import tpu as pltpu
from jax.experimental.pallas import tpu_sc as plsc

B = 4
E = 80000
M = B * E
C = 128

BH = 2
MH = BH * E

NW = 32
PER_W = MH // NW
KCH = 40
NCHUNK = PER_W // KCH
NGRP = 5
GCH = NCHUNK // NGRP

BLK = 3200
NEB = E // BLK


def _sc_gather_body(table, idxw, o1, o2, o3, o4, ivall,
                    b00, b01, b02, b03, b10, b11, b12, b13,
                    sg0, sg1, sw0, sw1):
    wid = lax.axis_index("s") * 2 + lax.axis_index("c")
    base0 = wid * PER_W

    bufs = ((b00, b01, b02, b03), (b10, b11, b12, b13))
    sgs = (sg0, sg1)
    sws = (sw0, sw1)
    ohs = (o1, o2, o3, o4)

    def group(g, carry):
        pltpu.sync_copy(idxw.at[wid, g], ivall)

        def fire_gathers(lci, s):
            return [pltpu.async_copy(table.at[ivall.at[lci * 4 + j]],
                                     bufs[s][j], sgs[s]) for j in range(4)]

        def fire_writes(lci, s):
            base = base0 + (g * GCH + lci) * KCH
            for j in range(4):
                pltpu.async_copy(bufs[s][j], ohs[j].at[pl.ds(base, KCH)],
                                 sws[s])

        def wait_writes(s):
            for j in range(4):
                pltpu.make_async_copy(bufs[s][j], ohs[j].at[pl.ds(0, KCH)],
                                      sws[s]).wait()

        def run_chunk(lci, s):
            cps = fire_gathers(lci, s)
            for cp in cps:
                cp.wait()
            fire_writes(lci, s)

        run_chunk(0, 0)
        run_chunk(1, 1)

        def pair(k, c2):
            for s in (0, 1):
                wait_writes(s)
                run_chunk(2 + 2 * k + s, s)
            return c2

        lax.fori_loop(0, (GCH - 2) // 2, pair, 0)
        if (GCH - 2) % 2 == 1:
            wait_writes(0)
            run_chunk(GCH - 1, 0)
        wait_writes(0)
        wait_writes(1)
        return carry

    lax.fori_loop(0, NGRP, group, 0)


def _make_sc_gather(dtype):
    mesh = plsc.VectorSubcoreMesh(core_axis_name="c", subcore_axis_name="s")
    out_t = [jax.ShapeDtypeStruct((MH, C), dtype)] * 4
    scratch = ([pltpu.VMEM((GCH * 4, KCH), jnp.int32)]
               + [pltpu.VMEM((KCH, C), dtype)] * 8
               + [pltpu.SemaphoreType.DMA] * 4)
    return pl.kernel(_sc_gather_body, mesh=mesh, out_type=out_t,
                     scratch_types=scratch)


_HI = np.uint32(0xFFFF0000)


def _pack2(a, b):
    ah = lax.bitcast_convert_type(a.astype(jnp.bfloat16).astype(jnp.float32),
                                  jnp.uint32)
    bh = lax.bitcast_convert_type(b.astype(jnp.bfloat16).astype(jnp.float32),
                                  jnp.uint32)
    return lax.bitcast_convert_type((ah & _HI) | (bh >> 16), jnp.int32)


def _unpack_hi(p):
    u = lax.bitcast_convert_type(p, jnp.uint32)
    return lax.bitcast_convert_type(u & _HI, jnp.float32)


def _unpack_lo(p):
    u = lax.bitcast_convert_type(p, jnp.uint32)
    return lax.bitcast_convert_type(u << 16, jnp.float32)


def _combo(a1, a2, a3, a4):
    return [a1 + a3, a2 + a4, jnp.abs(a1 - a3), jnp.abs(a2 - a4)]


def _bf(xs):
    return [x.astype(jnp.bfloat16) for x in xs]


def _pack_body(fu, fd, t1):
    t1[...] = _pack2(fu[...], fd[...])


def _conv1_body(fu, fd, p1, p2, p3, p4, w1, wz, b1, bz, y1, z):
    hi = [_unpack_hi(p[...]) for p in (p1, p2, p3, p4)]
    lo = [_unpack_lo(p[...]) for p in (p1, p2, p3, p4)]
    gy = jnp.concatenate([fu[...].astype(jnp.bfloat16)] + _bf(_combo(*hi)),
                         axis=1)
    gz = jnp.concatenate([fd[...].astype(jnp.bfloat16)] + _bf(_combo(*lo)),
                         axis=1)
    y1[...] = jnp.dot(gy, w1[...], preferred_element_type=jnp.float32) + b1[...]
    z[...] = jnp.dot(gz, wz[...], preferred_element_type=jnp.float32) + bz[...]


def _stats_epilogue(i, y, acc1, acc2, scale, shift):
    @pl.when(i == 0)
    def _():
        acc1[...] = jnp.zeros_like(acc1)
        acc2[...] = jnp.zeros_like(acc2)

    acc1[...] += jnp.sum(y, axis=0, keepdims=True)
    acc2[...] += jnp.sum(y * y, axis=0, keepdims=True)

    @pl.when(i == NEB - 1)
    def _():
        mean = acc1[...] * (1.0 / E)
        var = acc2[...] * (1.0 / E) - mean * mean
        rstd = lax.rsqrt(var + 1e-5)
        scale[...] = rstd.reshape(1, 1, C)
        shift[...] = (-mean * rstd).reshape(1, 1, C)


def _conv2_body(y1r, q1, q2, q3, q4, z, w, y2, scale, shift, acc1, acc2):
    i = pl.program_id(1)
    g = jnp.concatenate([y1r[...].astype(jnp.bfloat16)]
                        + _bf(_combo(q1[...], q2[...], q3[...], q4[...])),
                        axis=1)
    y = jnp.dot(g, w[...], preferred_element_type=jnp.float32) + z[...]
    y2[...] = y.astype(jnp.bfloat16)
    _stats_epilogue(i, y, acc1, acc2, scale, shift)


def _norm_body(y2r, scale, shift, x1):
    x1[...] = jnp.maximum(y2r[...] * scale[...].reshape(1, C)
                          + shift[...].reshape(1, C), 0.0)


def _conv3_body(x1r, a1, a2, a3, a4, w, bias, y3, scale, shift, acc1, acc2):
    i = pl.program_id(1)
    g = jnp.concatenate([x1r[...].astype(jnp.bfloat16)]
                        + _bf(_combo(a1[...], a2[...], a3[...], a4[...])),
                        axis=1)
    y = jnp.dot(g, w[...], preferred_element_type=jnp.float32) + bias[...]
    y3[...] = y.astype(jnp.bfloat16)
    _stats_epilogue(i, y, acc1, acc2, scale, shift)


def _final_body(y3r, x1r, scale, shift, out):
    r = jnp.maximum(y3r[...] * scale[...].reshape(1, C)
                    + shift[...].reshape(1, C) + x1r[...], 0.0)
    out[...] = r.T.reshape(1, C, BLK)


def _row1():
    return pl.BlockSpec((BLK, C), lambda i: (i, 0))


def _row1_off(h):
    off = h * (MH // BLK)
    return pl.BlockSpec((BLK, C), lambda i, o=off: (i + o, 0))


def _row2():
    return pl.BlockSpec((BLK, C), lambda b, i: (b * NEB + i, 0))


def _w1_spec(k):
    return pl.BlockSpec((k, C), lambda i: (0, 0))


def _w2_spec(k):
    return pl.BlockSpec((k, C), lambda b, i: (0, 0))


def _stat_spec():
    return pl.BlockSpec((1, 1, C), lambda b, i: (b, 0, 0))


_STAT_SHAPE = jax.ShapeDtypeStruct((BH, 1, C), jnp.float32)
_ROW_F32 = jax.ShapeDtypeStruct((MH, C), jnp.float32)
_ROW_BF16 = jax.ShapeDtypeStruct((MH, C), jnp.bfloat16)


def _pack_call(h, fu, fd):
    return pl.pallas_call(
        _pack_body, grid=(MH // BLK,),
        in_specs=[_row1_off(h), _row1_off(h)], out_specs=_row1(),
        out_shape=jax.ShapeDtypeStruct((MH, C), jnp.int32),
    )(fu, fd)


def _conv1_call(h, fu, fd, p, w1, wz, b1, bz):
    return pl.pallas_call(
        _conv1_body, grid=(MH // BLK,),
        in_specs=[_row1_off(h)] * 2 + [_row1()] * 4
                 + [_w1_spec(5 * C), _w1_spec(5 * C),
                    _w1_spec(1), _w1_spec(1)],
        out_specs=[_row1(), _row1()],
        out_shape=[_ROW_F32, _ROW_F32],
    )(fu, fd, *p, w1, wz, b1, bz)


def _conv2_call(y1, q, z, wc):
    return pl.pallas_call(
        _conv2_body, grid=(BH, NEB),
        in_specs=[_row2()] * 6 + [_w2_spec(5 * C)],
        out_specs=[_row2(), _stat_spec(), _stat_spec()],
        out_shape=[_ROW_BF16, _STAT_SHAPE, _STAT_SHAPE],
        scratch_shapes=[pltpu.VMEM((1, C), jnp.float32),
                        pltpu.VMEM((1, C), jnp.float32)],
    )(y1, *q, z, wc)


def _norm_call(y2, scale, shift):
    return pl.pallas_call(
        _norm_body, grid=(BH, NEB),
        in_specs=[_row2(), _stat_spec(), _stat_spec()],
        out_specs=_row2(),
        out_shape=_ROW_F32,
    )(y2, scale, shift)


def _conv3_call(x1, a, wc, bias):
    return pl.pallas_call(
        _conv3_body, grid=(BH, NEB),
        in_specs=[_row2()] * 5 + [_w2_spec(5 * C), _w2_spec(1)],
        out_specs=[_row2(), _stat_spec(), _stat_spec()],
        out_shape=[_ROW_BF16, _STAT_SHAPE, _STAT_SHAPE],
        scratch_shapes=[pltpu.VMEM((1, C), jnp.float32),
                        pltpu.VMEM((1, C), jnp.float32)],
    )(x1, *a, wc, bias)


def _final_call(y3, x1, scale, shift):
    return pl.pallas_call(
        _final_body, grid=(BH, NEB),
        in_specs=[_row2(), _row2(), _stat_spec(), _stat_spec()],
        out_specs=pl.BlockSpec((1, C, BLK), lambda b, i: (b, 0, i)),
        out_shape=jax.ShapeDtypeStruct((BH, C, E), jnp.float32),
    )(y3, x1, scale, shift)


def kernel(from_up, from_down, gemm_edges, W_up, b_up, W1, b1, W2, b2):
    def wcat(W, cols):
        return jnp.concatenate([W[:, cs, k].T for (cs, k) in cols],
                               axis=0).astype(jnp.bfloat16)

    full = slice(0, C)
    lo, hi = slice(0, C), slice(C, 2 * C)
    taps5 = [0, 1, 2, 3, 4]
    wc1 = wcat(W_up, [(full, k) for k in taps5])
    wcz = wcat(W1, [(hi, k) for k in taps5])
    wc2 = wcat(W1, [(lo, k) for k in taps5])
    wc3 = wcat(W2, [(full, k) for k in taps5])
    bu = b_up.reshape(1, C)
    bz = b1.reshape(1, C)
    b2r = b2.reshape(1, C)

    sc_i = _make_sc_gather(jnp.int32)
    sc_f = _make_sc_gather(jnp.float32)

    H = B // BH
    fu = from_up.transpose(0, 2, 1).reshape(M, C)
    fd = from_down.transpose(0, 2, 1).reshape(M, C)
    idxw = []
    for h in range(H):
        sl = slice(h * BH, (h + 1) * BH)
        ge = (gemm_edges[sl].astype(jnp.int32)
              + (jnp.arange(BH, dtype=jnp.int32) * E)[:, None, None])
        idxw.append(ge.reshape(MH, 4).T
                    .reshape(4, NW, NGRP, GCH, KCH)
                    .transpose(1, 2, 3, 0, 4)
                    .reshape(NW, NGRP, GCH * 4, KCH))

    t1 = [_pack_call(h, fu, fd) for h in range(H)]
    p = [sc_i(t1[h], idxw[h]) for h in range(H)]
    y1z = [_conv1_call(h, fu, fd, p[h], wc1, wcz, bu, bz) for h in range(H)]
    q = [sc_f(y1z[h][0], idxw[h]) for h in range(H)]
    y2s = [_conv2_call(y1z[h][0], q[h], y1z[h][1], wc2) for h in range(H)]
    x1 = [_norm_call(*y2s[h]) for h in range(H)]
    a = [sc_f(x1[h], idxw[h]) for h in range(H)]
    y3s = [_conv3_call(x1[h], a[h], wc3, b2r) for h in range(H)]
    out = [_final_call(y3s[h][0], x1[h], y3s[h][1], y3s[h][2])
           for h in range(H)]
    return jnp.concatenate(out, axis=0)


# speedup vs baseline: 3.7417x; 1.0242x over previous
import jax
import jax.numpy as jnp
import numpy as np
from jax import lax
from jax.experimental import pallas as pl
from jax.experimental.pallas import tpu as pltpu
from jax.experimental.pallas import tpu_sc as plsc

B = 4
E = 80000
M = B * E
C = 128

BH = 2
MH = BH * E

NW = 32
PER_W = MH // NW
KCH = 40
NCHUNK = PER_W // KCH
NGRP = 5
GCH = NCHUNK // NGRP

BLK = 3200
NEB = E // BLK


def _sc_gather_body(table, idxw, o1, o2, o3, o4, ivall,
                    b00, b01, b02, b03, b10, b11, b12, b13,
                    sg0, sg1, sw0, sw1):
    wid = lax.axis_index("s") * 2 + lax.axis_index("c")
    base0 = wid * PER_W

    bufs = ((b00, b01, b02, b03), (b10, b11, b12, b13))
    sgs = (sg0, sg1)
    sws = (sw0, sw1)
    ohs = (o1, o2, o3, o4)

    def group(g, carry):
        pltpu.sync_copy(idxw.at[wid, g], ivall)

        def fire_gathers(lci, s):
            return [pltpu.async_copy(table.at[ivall.at[lci * 4 + j]],
                                     bufs[s][j], sgs[s]) for j in range(4)]

        def fire_writes(lci, s):
            base = base0 + (g * GCH + lci) * KCH
            for j in range(4):
                pltpu.async_copy(bufs[s][j], ohs[j].at[pl.ds(base, KCH)],
                                 sws[s])

        def wait_writes(s):
            for j in range(4):
                pltpu.make_async_copy(bufs[s][j], ohs[j].at[pl.ds(0, KCH)],
                                      sws[s]).wait()

        def run_chunk(lci, s):
            cps = fire_gathers(lci, s)
            for cp in cps:
                cp.wait()
            fire_writes(lci, s)

        run_chunk(0, 0)
        run_chunk(1, 1)

        def pair(k, c2):
            for s in (0, 1):
                wait_writes(s)
                run_chunk(2 + 2 * k + s, s)
            return c2

        lax.fori_loop(0, (GCH - 2) // 2, pair, 0)
        if (GCH - 2) % 2 == 1:
            wait_writes(0)
            run_chunk(GCH - 1, 0)
        wait_writes(0)
        wait_writes(1)
        return carry

    lax.fori_loop(0, NGRP, group, 0)


def _make_sc_gather(dtype):
    mesh = plsc.VectorSubcoreMesh(core_axis_name="c", subcore_axis_name="s")
    out_t = [jax.ShapeDtypeStruct((MH, C), dtype)] * 4
    scratch = ([pltpu.VMEM((GCH * 4, KCH), jnp.int32)]
               + [pltpu.VMEM((KCH, C), dtype)] * 8
               + [pltpu.SemaphoreType.DMA] * 4)
    return pl.kernel(_sc_gather_body, mesh=mesh, out_type=out_t,
                     scratch_types=scratch)


_HI = np.uint32(0xFFFF0000)


def _pack2(a, b):
    ah = lax.bitcast_convert_type(a.astype(jnp.bfloat16).astype(jnp.float32),
                                  jnp.uint32)
    bh = lax.bitcast_convert_type(b.astype(jnp.bfloat16).astype(jnp.float32),
                                  jnp.uint32)
    return lax.bitcast_convert_type((ah & _HI) | (bh >> 16), jnp.int32)


def _unpack_hi(p):
    u = lax.bitcast_convert_type(p, jnp.uint32)
    return lax.bitcast_convert_type(u & _HI, jnp.float32)


def _unpack_lo(p):
    u = lax.bitcast_convert_type(p, jnp.uint32)
    return lax.bitcast_convert_type(u << 16, jnp.float32)


def _combo(a1, a2, a3, a4):
    return [a1 + a3, a2 + a4, jnp.abs(a1 - a3), jnp.abs(a2 - a4)]


def _bf(xs):
    return [x.astype(jnp.bfloat16) for x in xs]


def _pack_body(fu, fd, t1):
    t1[...] = _pack2(fu[...], fd[...])


def _conv1_body(fu, fd, p1, p2, p3, p4, w1, wz, b1, bz, y1, z):
    hi = [_unpack_hi(p[...]) for p in (p1, p2, p3, p4)]
    lo = [_unpack_lo(p[...]) for p in (p1, p2, p3, p4)]
    gy = jnp.concatenate([fu[...].astype(jnp.bfloat16)] + _bf(_combo(*hi)),
                         axis=1)
    gz = jnp.concatenate([fd[...].astype(jnp.bfloat16)] + _bf(_combo(*lo)),
                         axis=1)
    y1[...] = jnp.dot(gy, w1[...], preferred_element_type=jnp.float32) + b1[...]
    z[...] = jnp.dot(gz, wz[...], preferred_element_type=jnp.float32) + bz[...]


def _stats_epilogue(i, y, acc1, acc2, scale, shift):
    @pl.when(i == 0)
    def _():
        acc1[...] = jnp.zeros_like(acc1)
        acc2[...] = jnp.zeros_like(acc2)

    acc1[...] += jnp.sum(y, axis=0, keepdims=True)
    acc2[...] += jnp.sum(y * y, axis=0, keepdims=True)

    @pl.when(i == NEB - 1)
    def _():
        mean = acc1[...] * (1.0 / E)
        var = acc2[...] * (1.0 / E) - mean * mean
        rstd = lax.rsqrt(var + 1e-5)
        scale[...] = rstd.reshape(1, 1, C)
        shift[...] = (-mean * rstd).reshape(1, 1, C)


def _conv2_body(y1r, q1, q2, q3, q4, z, w, y2, scale, shift, acc1, acc2):
    i = pl.program_id(1)
    g = jnp.concatenate([y1r[...].astype(jnp.bfloat16)]
                        + _bf(_combo(q1[...], q2[...], q3[...], q4[...])),
                        axis=1)
    y = jnp.dot(g, w[...], preferred_element_type=jnp.float32) + z[...]
    y2[...] = y
    _stats_epilogue(i, y, acc1, acc2, scale, shift)


def _x1(y2val, s2, h2):
    return jnp.maximum(y2val * s2.reshape(1, C) + h2.reshape(1, C), 0.0)


def _conv3_body(y2r, a1, a2, a3, a4, s2, h2, w, bias,
                y3, scale, shift, acc1, acc2):
    i = pl.program_id(1)
    x0 = _x1(y2r[...], s2[...], h2[...])
    xs = [_x1(a[...], s2[...], h2[...]) for a in (a1, a2, a3, a4)]
    g = jnp.concatenate([x0.astype(jnp.bfloat16)] + _bf(_combo(*xs)), axis=1)
    y = jnp.dot(g, w[...], preferred_element_type=jnp.float32) + bias[...]
    y3[...] = y.astype(jnp.bfloat16)
    _stats_epilogue(i, y, acc1, acc2, scale, shift)


def _final_body(y3r, y2r, s2, h2, scale, shift, out):
    r = jnp.maximum(y3r[...] * scale[...].reshape(1, C)
                    + shift[...].reshape(1, C)
                    + _x1(y2r[...], s2[...], h2[...]), 0.0)
    out[...] = r.T.reshape(1, C, BLK)


def _row1():
    return pl.BlockSpec((BLK, C), lambda i: (i, 0))


def _row1_off(h):
    off = h * (MH // BLK)
    return pl.BlockSpec((BLK, C), lambda i, o=off: (i + o, 0))


def _row2():
    return pl.BlockSpec((BLK, C), lambda b, i: (b * NEB + i, 0))


def _w1_spec(k):
    return pl.BlockSpec((k, C), lambda i: (0, 0))


def _w2_spec(k):
    return pl.BlockSpec((k, C), lambda b, i: (0, 0))


def _stat_spec():
    return pl.BlockSpec((1, 1, C), lambda b, i: (b, 0, 0))


_STAT_SHAPE = jax.ShapeDtypeStruct((BH, 1, C), jnp.float32)
_ROW_F32 = jax.ShapeDtypeStruct((MH, C), jnp.float32)
_ROW_BF16 = jax.ShapeDtypeStruct((MH, C), jnp.bfloat16)


def _pack_call(h, fu, fd):
    return pl.pallas_call(
        _pack_body, grid=(MH // BLK,),
        in_specs=[_row1_off(h), _row1_off(h)], out_specs=_row1(),
        out_shape=jax.ShapeDtypeStruct((MH, C), jnp.int32),
    )(fu, fd)


def _conv1_call(h, fu, fd, p, w1, wz, b1, bz):
    return pl.pallas_call(
        _conv1_body, grid=(MH // BLK,),
        in_specs=[_row1_off(h)] * 2 + [_row1()] * 4
                 + [_w1_spec(5 * C), _w1_spec(5 * C),
                    _w1_spec(1), _w1_spec(1)],
        out_specs=[_row1(), _row1()],
        out_shape=[_ROW_F32, _ROW_F32],
    )(fu, fd, *p, w1, wz, b1, bz)


def _conv2_call(y1, q, z, wc):
    return pl.pallas_call(
        _conv2_body, grid=(BH, NEB),
        in_specs=[_row2()] * 6 + [_w2_spec(5 * C)],
        out_specs=[_row2(), _stat_spec(), _stat_spec()],
        out_shape=[_ROW_F32, _STAT_SHAPE, _STAT_SHAPE],
        scratch_shapes=[pltpu.VMEM((1, C), jnp.float32),
                        pltpu.VMEM((1, C), jnp.float32)],
    )(y1, *q, z, wc)


def _conv3_call(y2, a, s2, h2, wc, bias):
    return pl.pallas_call(
        _conv3_body, grid=(BH, NEB),
        in_specs=([_row2()] * 5 + [_stat_spec(), _stat_spec()]
                  + [_w2_spec(5 * C), _w2_spec(1)]),
        out_specs=[_row2(), _stat_spec(), _stat_spec()],
        out_shape=[_ROW_BF16, _STAT_SHAPE, _STAT_SHAPE],
        scratch_shapes=[pltpu.VMEM((1, C), jnp.float32),
                        pltpu.VMEM((1, C), jnp.float32)],
    )(y2, *a, s2, h2, wc, bias)


def _final_call(y3, y2, s2, h2, scale, shift):
    return pl.pallas_call(
        _final_body, grid=(BH, NEB),
        in_specs=[_row2(), _row2(), _stat_spec(), _stat_spec(),
                  _stat_spec(), _stat_spec()],
        out_specs=pl.BlockSpec((1, C, BLK), lambda b, i: (b, 0, i)),
        out_shape=jax.ShapeDtypeStruct((BH, C, E), jnp.float32),
    )(y3, y2, s2, h2, scale, shift)


def kernel(from_up, from_down, gemm_edges, W_up, b_up, W1, b1, W2, b2):
    def wcat(W, cols):
        return jnp.concatenate([W[:, cs, k].T for (cs, k) in cols],
                               axis=0).astype(jnp.bfloat16)

    full = slice(0, C)
    lo, hi = slice(0, C), slice(C, 2 * C)
    taps5 = [0, 1, 2, 3, 4]
    wc1 = wcat(W_up, [(full, k) for k in taps5])
    wcz = wcat(W1, [(hi, k) for k in taps5])
    wc2 = wcat(W1, [(lo, k) for k in taps5])
    wc3 = wcat(W2, [(full, k) for k in taps5])
    bu = b_up.reshape(1, C)
    bz = b1.reshape(1, C)
    b2r = b2.reshape(1, C)

    sc_i = _make_sc_gather(jnp.int32)
    sc_f = _make_sc_gather(jnp.float32)

    H = B // BH
    fu = from_up.transpose(0, 2, 1).reshape(M, C)
    fd = from_down.transpose(0, 2, 1).reshape(M, C)
    idxw = []
    for h in range(H):
        sl = slice(h * BH, (h + 1) * BH)
        ge = (gemm_edges[sl].astype(jnp.int32)
              + (jnp.arange(BH, dtype=jnp.int32) * E)[:, None, None])
        idxw.append(ge.reshape(MH, 4).T
                    .reshape(4, NW, NGRP, GCH, KCH)
                    .transpose(1, 2, 3, 0, 4)
                    .reshape(NW, NGRP, GCH * 4, KCH))

    t1 = [_pack_call(h, fu, fd) for h in range(H)]
    p = [sc_i(t1[h], idxw[h]) for h in range(H)]
    y1z = [_conv1_call(h, fu, fd, p[h], wc1, wcz, bu, bz) for h in range(H)]
    q = [sc_f(y1z[h][0], idxw[h]) for h in range(H)]
    y2s = [_conv2_call(y1z[h][0], q[h], y1z[h][1], wc2) for h in range(H)]
    a = [sc_f(y2s[h][0], idxw[h]) for h in range(H)]
    y3s = [_conv3_call(y2s[h][0], a[h], y2s[h][1], y2s[h][2], wc3, b2r)
           for h in range(H)]
    out = [_final_call(y3s[h][0], y2s[h][0], y2s[h][1], y2s[h][2],
                       y3s[h][1], y3s[h][2]) for h in range(H)]
    return jnp.concatenate(out, axis=0)


# speedup vs baseline: 3.8787x; 1.0366x over previous
import jax
import jax.numpy as jnp
import numpy as np
from jax import lax
from jax.experimental import pallas as pl
from jax.experimental.pallas import tpu as pltpu
from jax.experimental.pallas import tpu_sc as plsc

B = 4
E = 80000
M = B * E
C = 128

BH = 2
MH = BH * E

NW = 32
PER_W = MH // NW
KCH = 40
NCHUNK = PER_W // KCH
NGRP = 5
GCH = NCHUNK // NGRP

BLK = 3200
NEB = E // BLK


def _sc_gather_body(table, idxw, o1, o2, o3, o4, ivall,
                    b00, b01, b02, b03, b10, b11, b12, b13,
                    sg0, sg1, sw0, sw1):
    wid = lax.axis_index("s") * 2 + lax.axis_index("c")
    base0 = wid * PER_W

    bufs = ((b00, b01, b02, b03), (b10, b11, b12, b13))
    sgs = (sg0, sg1)
    sws = (sw0, sw1)
    ohs = (o1, o2, o3, o4)

    def group(g, carry):
        pltpu.sync_copy(idxw.at[wid, g], ivall)

        def fire_gathers(lci, s):
            return [pltpu.async_copy(table.at[ivall.at[lci * 4 + j]],
                                     bufs[s][j], sgs[s]) for j in range(4)]

        def fire_writes(lci, s):
            base = base0 + (g * GCH + lci) * KCH
            for j in range(4):
                pltpu.async_copy(bufs[s][j], ohs[j].at[pl.ds(base, KCH)],
                                 sws[s])

        def wait_writes(s):
            for j in range(4):
                pltpu.make_async_copy(bufs[s][j], ohs[j].at[pl.ds(0, KCH)],
                                      sws[s]).wait()

        def run_chunk(lci, s):
            cps = fire_gathers(lci, s)
            for cp in cps:
                cp.wait()
            fire_writes(lci, s)

        run_chunk(0, 0)
        run_chunk(1, 1)

        def pair(k, c2):
            for s in (0, 1):
                wait_writes(s)
                run_chunk(2 + 2 * k + s, s)
            return c2

        lax.fori_loop(0, (GCH - 2) // 2, pair, 0)
        if (GCH - 2) % 2 == 1:
            wait_writes(0)
            run_chunk(GCH - 1, 0)
        wait_writes(0)
        wait_writes(1)
        return carry

    lax.fori_loop(0, NGRP, group, 0)


def _make_sc_gather(dtype):
    mesh = plsc.VectorSubcoreMesh(core_axis_name="c", subcore_axis_name="s")
    out_t = [jax.ShapeDtypeStruct((MH, C), dtype)] * 4
    scratch = ([pltpu.VMEM((GCH * 4, KCH), jnp.int32)]
               + [pltpu.VMEM((KCH, C), dtype)] * 8
               + [pltpu.SemaphoreType.DMA] * 4)
    return pl.kernel(_sc_gather_body, mesh=mesh, out_type=out_t,
                     scratch_types=scratch)


_HI = np.uint32(0xFFFF0000)


def _pack2(a, b):
    ah = lax.bitcast_convert_type(a.astype(jnp.bfloat16).astype(jnp.float32),
                                  jnp.uint32)
    bh = lax.bitcast_convert_type(b.astype(jnp.bfloat16).astype(jnp.float32),
                                  jnp.uint32)
    return lax.bitcast_convert_type((ah & _HI) | (bh >> 16), jnp.int32)


def _unpack_hi(p):
    u = lax.bitcast_convert_type(p, jnp.uint32)
    return lax.bitcast_convert_type(u & _HI, jnp.float32)


def _unpack_lo(p):
    u = lax.bitcast_convert_type(p, jnp.uint32)
    return lax.bitcast_convert_type(u << 16, jnp.float32)


def _combo(a1, a2, a3, a4):
    return [a1 + a3, a2 + a4, jnp.abs(a1 - a3), jnp.abs(a2 - a4)]


def _bf(xs):
    return [x.astype(jnp.bfloat16) for x in xs]


def _pack_body(up, dn, t1, fuo, fdo):
    u = up[...].reshape(C, BLK).T
    d = dn[...].reshape(C, BLK).T
    t1[...] = _pack2(u, d)
    fuo[...] = u
    fdo[...] = d


def _conv1_body(fu, fd, p1, p2, p3, p4, w1, wz, b1, bz, y1, z):
    hi = [_unpack_hi(p[...]) for p in (p1, p2, p3, p4)]
    lo = [_unpack_lo(p[...]) for p in (p1, p2, p3, p4)]
    gy = jnp.concatenate([fu[...].astype(jnp.bfloat16)] + _bf(_combo(*hi)),
                         axis=1)
    gz = jnp.concatenate([fd[...].astype(jnp.bfloat16)] + _bf(_combo(*lo)),
                         axis=1)
    y1[...] = jnp.dot(gy, w1[...], preferred_element_type=jnp.float32) + b1[...]
    z[...] = jnp.dot(gz, wz[...], preferred_element_type=jnp.float32) + bz[...]


def _stats_epilogue(i, y, acc1, acc2, scale, shift):
    @pl.when(i == 0)
    def _():
        acc1[...] = jnp.zeros_like(acc1)
        acc2[...] = jnp.zeros_like(acc2)

    acc1[...] += jnp.sum(y, axis=0, keepdims=True)
    acc2[...] += jnp.sum(y * y, axis=0, keepdims=True)

    @pl.when(i == NEB - 1)
    def _():
        mean = acc1[...] * (1.0 / E)
        var = acc2[...] * (1.0 / E) - mean * mean
        rstd = lax.rsqrt(var + 1e-5)
        scale[...] = rstd.reshape(1, 1, C)
        shift[...] = (-mean * rstd).reshape(1, 1, C)


def _conv2_body(y1r, q1, q2, q3, q4, z, w, y2, scale, shift, acc1, acc2):
    i = pl.program_id(1)
    g = jnp.concatenate([y1r[...].astype(jnp.bfloat16)]
                        + _bf(_combo(q1[...], q2[...], q3[...], q4[...])),
                        axis=1)
    y = jnp.dot(g, w[...], preferred_element_type=jnp.float32) + z[...]
    y2[...] = y
    _stats_epilogue(i, y, acc1, acc2, scale, shift)


def _x1(y2val, s2, h2):
    return jnp.maximum(y2val * s2.reshape(1, C) + h2.reshape(1, C), 0.0)


def _conv3_body(y2r, a1, a2, a3, a4, s2, h2, w, bias,
                y3, scale, shift, acc1, acc2):
    i = pl.program_id(1)
    x0 = _x1(y2r[...], s2[...], h2[...])
    xs = [_x1(a[...], s2[...], h2[...]) for a in (a1, a2, a3, a4)]
    g = jnp.concatenate([x0.astype(jnp.bfloat16)] + _bf(_combo(*xs)), axis=1)
    y = jnp.dot(g, w[...], preferred_element_type=jnp.float32) + bias[...]
    y3[...] = y.astype(jnp.bfloat16)
    _stats_epilogue(i, y, acc1, acc2, scale, shift)


def _final_body(y3r, y2r, s2, h2, scale, shift, out):
    r = jnp.maximum(y3r[...] * scale[...].reshape(1, C)
                    + shift[...].reshape(1, C)
                    + _x1(y2r[...], s2[...], h2[...]), 0.0)
    out[...] = r.T.reshape(1, C, BLK)


def _row1():
    return pl.BlockSpec((BLK, C), lambda i: (i, 0))


def _row1_off(h):
    off = h * (MH // BLK)
    return pl.BlockSpec((BLK, C), lambda i, o=off: (i + o, 0))


def _row2():
    return pl.BlockSpec((BLK, C), lambda b, i: (b * NEB + i, 0))


def _w1_spec(k):
    return pl.BlockSpec((k, C), lambda i: (0, 0))


def _w2_spec(k):
    return pl.BlockSpec((k, C), lambda b, i: (0, 0))


def _stat_spec():
    return pl.BlockSpec((1, 1, C), lambda b, i: (b, 0, 0))


_STAT_SHAPE = jax.ShapeDtypeStruct((BH, 1, C), jnp.float32)
_ROW_F32 = jax.ShapeDtypeStruct((MH, C), jnp.float32)
_ROW_BF16 = jax.ShapeDtypeStruct((MH, C), jnp.bfloat16)


def _pack_call(h, from_up, from_down):
    spec = pl.BlockSpec((1, C, BLK),
                        lambda i, hh=h: (hh * BH + i // NEB, 0, i % NEB))
    return pl.pallas_call(
        _pack_body, grid=(MH // BLK,),
        in_specs=[spec, spec],
        out_specs=[_row1(), _row1(), _row1()],
        out_shape=[jax.ShapeDtypeStruct((MH, C), jnp.int32),
                   _ROW_F32, _ROW_F32],
    )(from_up, from_down)


def _conv1_call(fu, fd, p, w1, wz, b1, bz):
    return pl.pallas_call(
        _conv1_body, grid=(MH // BLK,),
        in_specs=[_row1()] * 6
                 + [_w1_spec(5 * C), _w1_spec(5 * C),
                    _w1_spec(1), _w1_spec(1)],
        out_specs=[_row1(), _row1()],
        out_shape=[_ROW_F32, _ROW_F32],
    )(fu, fd, *p, w1, wz, b1, bz)


def _conv2_call(y1, q, z, wc):
    return pl.pallas_call(
        _conv2_body, grid=(BH, NEB),
        in_specs=[_row2()] * 6 + [_w2_spec(5 * C)],
        out_specs=[_row2(), _stat_spec(), _stat_spec()],
        out_shape=[_ROW_F32, _STAT_SHAPE, _STAT_SHAPE],
        scratch_shapes=[pltpu.VMEM((1, C), jnp.float32),
                        pltpu.VMEM((1, C), jnp.float32)],
    )(y1, *q, z, wc)


def _conv3_call(y2, a, s2, h2, wc, bias):
    return pl.pallas_call(
        _conv3_body, grid=(BH, NEB),
        in_specs=([_row2()] * 5 + [_stat_spec(), _stat_spec()]
                  + [_w2_spec(5 * C), _w2_spec(1)]),
        out_specs=[_row2(), _stat_spec(), _stat_spec()],
        out_shape=[_ROW_BF16, _STAT_SHAPE, _STAT_SHAPE],
        scratch_shapes=[pltpu.VMEM((1, C), jnp.float32),
                        pltpu.VMEM((1, C), jnp.float32)],
    )(y2, *a, s2, h2, wc, bias)


def _final_call(y3, y2, s2, h2, scale, shift):
    return pl.pallas_call(
        _final_body, grid=(BH, NEB),
        in_specs=[_row2(), _row2(), _stat_spec(), _stat_spec(),
                  _stat_spec(), _stat_spec()],
        out_specs=pl.BlockSpec((1, C, BLK), lambda b, i: (b, 0, i)),
        out_shape=jax.ShapeDtypeStruct((BH, C, E), jnp.float32),
    )(y3, y2, s2, h2, scale, shift)


def kernel(from_up, from_down, gemm_edges, W_up, b_up, W1, b1, W2, b2):
    def wcat(W, cols):
        return jnp.concatenate([W[:, cs, k].T for (cs, k) in cols],
                               axis=0).astype(jnp.bfloat16)

    full = slice(0, C)
    lo, hi = slice(0, C), slice(C, 2 * C)
    taps5 = [0, 1, 2, 3, 4]
    wc1 = wcat(W_up, [(full, k) for k in taps5])
    wcz = wcat(W1, [(hi, k) for k in taps5])
    wc2 = wcat(W1, [(lo, k) for k in taps5])
    wc3 = wcat(W2, [(full, k) for k in taps5])
    bu = b_up.reshape(1, C)
    bz = b1.reshape(1, C)
    b2r = b2.reshape(1, C)

    sc_i = _make_sc_gather(jnp.int32)
    sc_f = _make_sc_gather(jnp.float32)

    H = B // BH
    idxw = []
    for h in range(H):
        sl = slice(h * BH, (h + 1) * BH)
        ge = (gemm_edges[sl].astype(jnp.int32)
              + (jnp.arange(BH, dtype=jnp.int32) * E)[:, None, None])
        idxw.append(ge.reshape(MH, 4).T
                    .reshape(4, NW, NGRP, GCH, KCH)
                    .transpose(1, 2, 3, 0, 4)
                    .reshape(NW, NGRP, GCH * 4, KCH))

    packed = [_pack_call(h, from_up, from_down) for h in range(H)]
    p = [sc_i(packed[h][0], idxw[h]) for h in range(H)]
    y1z = [_conv1_call(packed[h][1], packed[h][2], p[h], wc1, wcz, bu, bz)
           for h in range(H)]
    q = [sc_f(y1z[h][0], idxw[h]) for h in range(H)]
    y2s = [_conv2_call(y1z[h][0], q[h], y1z[h][1], wc2) for h in range(H)]
    a = [sc_f(y2s[h][0], idxw[h]) for h in range(H)]
    y3s = [_conv3_call(y2s[h][0], a[h], y2s[h][1], y2s[h][2], wc3, b2r)
           for h in range(H)]
    out = [_final_call(y3s[h][0], y2s[h][0], y2s[h][1], y2s[h][2],
                       y3s[h][1], y3s[h][2]) for h in range(H)]
    return jnp.concatenate(out, axis=0)


# speedup vs baseline: 4.0005x; 1.0314x over previous
import jax
import jax.numpy as jnp
import numpy as np
from jax import lax
from jax.experimental import pallas as pl
from jax.experimental.pallas import tpu as pltpu
from jax.experimental.pallas import tpu_sc as plsc

B = 4
E = 80000
M = B * E
C = 128

BH = 2
MH = BH * E

NW = 32
PER_W = MH // NW
KCH = 40
NCHUNK = PER_W // KCH
NGRP = 5
GCH = NCHUNK // NGRP

BLK = 3200
NEB = E // BLK


def _sc_gather_body(table, idxw, o1, o2, o3, o4, ivall,
                    b00, b01, b02, b03, b10, b11, b12, b13,
                    sg0, sg1, sw0, sw1):
    wid = lax.axis_index("s") * 2 + lax.axis_index("c")
    base0 = wid * PER_W

    bufs = ((b00, b01, b02, b03), (b10, b11, b12, b13))
    sgs = (sg0, sg1)
    sws = (sw0, sw1)
    ohs = (o1, o2, o3, o4)

    def group(g, carry):
        pltpu.sync_copy(idxw.at[wid, g], ivall)

        def fire_gathers(lci, s):
            return [pltpu.async_copy(table.at[ivall.at[lci * 4 + j]],
                                     bufs[s][j], sgs[s]) for j in range(4)]

        def fire_writes(lci, s):
            base = base0 + (g * GCH + lci) * KCH
            for j in range(4):
                pltpu.async_copy(bufs[s][j], ohs[j].at[pl.ds(base, KCH)],
                                 sws[s])

        def wait_writes(s):
            for j in range(4):
                pltpu.make_async_copy(bufs[s][j], ohs[j].at[pl.ds(0, KCH)],
                                      sws[s]).wait()

        def run_chunk(lci, s):
            cps = fire_gathers(lci, s)
            for cp in cps:
                cp.wait()
            fire_writes(lci, s)

        run_chunk(0, 0)
        run_chunk(1, 1)

        def pair(k, c2):
            for s in (0, 1):
                wait_writes(s)
                run_chunk(2 + 2 * k + s, s)
            return c2

        lax.fori_loop(0, (GCH - 2) // 2, pair, 0)
        if (GCH - 2) % 2 == 1:
            wait_writes(0)
            run_chunk(GCH - 1, 0)
        wait_writes(0)
        wait_writes(1)
        return carry

    lax.fori_loop(0, NGRP, group, 0)


def _make_sc_gather(dtype):
    mesh = plsc.VectorSubcoreMesh(core_axis_name="c", subcore_axis_name="s")
    out_t = [jax.ShapeDtypeStruct((MH, C), dtype)] * 4
    scratch = ([pltpu.VMEM((GCH * 4, KCH), jnp.int32)]
               + [pltpu.VMEM((KCH, C), dtype)] * 8
               + [pltpu.SemaphoreType.DMA] * 4)
    return pl.kernel(_sc_gather_body, mesh=mesh, out_type=out_t,
                     scratch_types=scratch)


_HI = np.uint32(0xFFFF0000)


def _pack2(a, b):
    ah = lax.bitcast_convert_type(a.astype(jnp.bfloat16).astype(jnp.float32),
                                  jnp.uint32)
    bh = lax.bitcast_convert_type(b.astype(jnp.bfloat16).astype(jnp.float32),
                                  jnp.uint32)
    return lax.bitcast_convert_type((ah & _HI) | (bh >> 16), jnp.int32)


def _unpack_hi(p):
    u = lax.bitcast_convert_type(p, jnp.uint32)
    return lax.bitcast_convert_type(u & _HI, jnp.float32)


def _unpack_lo(p):
    u = lax.bitcast_convert_type(p, jnp.uint32)
    return lax.bitcast_convert_type(u << 16, jnp.float32)


def _combo(a1, a2, a3, a4):
    return [a1 + a3, a2 + a4, jnp.abs(a1 - a3), jnp.abs(a2 - a4)]


def _bf(xs):
    return [x.astype(jnp.bfloat16) for x in xs]


def _pack_body(up, dn, t1, fuo, fdo):
    u = up[...].reshape(C, BLK).T
    d = dn[...].reshape(C, BLK).T
    t1[...] = _pack2(u, d)
    fuo[...] = u
    fdo[...] = d


def _conv1_body(fu, fd, p1, p2, p3, p4, w1, wz, b1, bz, t2):
    hi = [_unpack_hi(p[...]) for p in (p1, p2, p3, p4)]
    lo = [_unpack_lo(p[...]) for p in (p1, p2, p3, p4)]
    gy = jnp.concatenate([fu[...].astype(jnp.bfloat16)] + _bf(_combo(*hi)),
                         axis=1)
    gz = jnp.concatenate([fd[...].astype(jnp.bfloat16)] + _bf(_combo(*lo)),
                         axis=1)
    y1 = jnp.dot(gy, w1[...], preferred_element_type=jnp.float32) + b1[...]
    z = jnp.dot(gz, wz[...], preferred_element_type=jnp.float32) + bz[...]
    t2[...] = _pack2(y1, z)


def _stats_epilogue(i, y, acc1, acc2, scale, shift):
    @pl.when(i == 0)
    def _():
        acc1[...] = jnp.zeros_like(acc1)
        acc2[...] = jnp.zeros_like(acc2)

    acc1[...] += jnp.sum(y, axis=0, keepdims=True)
    acc2[...] += jnp.sum(y * y, axis=0, keepdims=True)

    @pl.when(i == NEB - 1)
    def _():
        mean = acc1[...] * (1.0 / E)
        var = acc2[...] * (1.0 / E) - mean * mean
        rstd = lax.rsqrt(var + 1e-5)
        scale[...] = rstd.reshape(1, 1, C)
        shift[...] = (-mean * rstd).reshape(1, 1, C)


def _conv2_body(t2r, q1, q2, q3, q4, w, y2, scale, shift, acc1, acc2):
    i = pl.program_id(1)
    y1f0 = _unpack_hi(t2r[...])
    z = _unpack_lo(t2r[...])
    qs = [_unpack_hi(q[...]) for q in (q1, q2, q3, q4)]
    g = jnp.concatenate([y1f0.astype(jnp.bfloat16)] + _bf(_combo(*qs)),
                        axis=1)
    y = jnp.dot(g, w[...], preferred_element_type=jnp.float32) + z
    y2[...] = y
    _stats_epilogue(i, y, acc1, acc2, scale, shift)


def _x1(y2val, s2, h2):
    return jnp.maximum(y2val * s2.reshape(1, C) + h2.reshape(1, C), 0.0)


def _conv3_body(y2r, a1, a2, a3, a4, s2, h2, w, bias,
                y3, scale, shift, acc1, acc2):
    i = pl.program_id(1)
    x0 = _x1(y2r[...], s2[...], h2[...])
    xs = [_x1(a[...], s2[...], h2[...]) for a in (a1, a2, a3, a4)]
    g = jnp.concatenate([x0.astype(jnp.bfloat16)] + _bf(_combo(*xs)), axis=1)
    y = jnp.dot(g, w[...], preferred_element_type=jnp.float32) + bias[...]
    y3[...] = y.astype(jnp.bfloat16)
    _stats_epilogue(i, y, acc1, acc2, scale, shift)


def _final_body(y3r, y2r, s2, h2, scale, shift, out):
    r = jnp.maximum(y3r[...] * scale[...].reshape(1, C)
                    + shift[...].reshape(1, C)
                    + _x1(y2r[...], s2[...], h2[...]), 0.0)
    out[...] = r.T.reshape(1, C, BLK)


def _row1():
    return pl.BlockSpec((BLK, C), lambda i: (i, 0))


def _row1_off(h):
    off = h * (MH // BLK)
    return pl.BlockSpec((BLK, C), lambda i, o=off: (i + o, 0))


def _row2():
    return pl.BlockSpec((BLK, C), lambda b, i: (b * NEB + i, 0))


def _w1_spec(k):
    return pl.BlockSpec((k, C), lambda i: (0, 0))


def _w2_spec(k):
    return pl.BlockSpec((k, C), lambda b, i: (0, 0))


def _stat_spec():
    return pl.BlockSpec((1, 1, C), lambda b, i: (b, 0, 0))


_STAT_SHAPE = jax.ShapeDtypeStruct((BH, 1, C), jnp.float32)
_ROW_F32 = jax.ShapeDtypeStruct((MH, C), jnp.float32)
_ROW_BF16 = jax.ShapeDtypeStruct((MH, C), jnp.bfloat16)


def _pack_call(h, from_up, from_down):
    spec = pl.BlockSpec((1, C, BLK),
                        lambda i, hh=h: (hh * BH + i // NEB, 0, i % NEB))
    return pl.pallas_call(
        _pack_body, grid=(MH // BLK,),
        in_specs=[spec, spec],
        out_specs=[_row1(), _row1(), _row1()],
        out_shape=[jax.ShapeDtypeStruct((MH, C), jnp.int32),
                   _ROW_F32, _ROW_F32],
    )(from_up, from_down)


def _conv1_call(fu, fd, p, w1, wz, b1, bz):
    return pl.pallas_call(
        _conv1_body, grid=(MH // BLK,),
        in_specs=[_row1()] * 6
                 + [_w1_spec(5 * C), _w1_spec(5 * C),
                    _w1_spec(1), _w1_spec(1)],
        out_specs=_row1(),
        out_shape=jax.ShapeDtypeStruct((MH, C), jnp.int32),
    )(fu, fd, *p, w1, wz, b1, bz)


def _conv2_call(t2, q, wc):
    return pl.pallas_call(
        _conv2_body, grid=(BH, NEB),
        in_specs=[_row2()] * 5 + [_w2_spec(5 * C)],
        out_specs=[_row2(), _stat_spec(), _stat_spec()],
        out_shape=[_ROW_F32, _STAT_SHAPE, _STAT_SHAPE],
        scratch_shapes=[pltpu.VMEM((1, C), jnp.float32),
                        pltpu.VMEM((1, C), jnp.float32)],
    )(t2, *q, wc)


def _conv3_call(y2, a, s2, h2, wc, bias):
    return pl.pallas_call(
        _conv3_body, grid=(BH, NEB),
        in_specs=([_row2()] * 5 + [_stat_spec(), _stat_spec()]
                  + [_w2_spec(5 * C), _w2_spec(1)]),
        out_specs=[_row2(), _stat_spec(), _stat_spec()],
        out_shape=[_ROW_BF16, _STAT_SHAPE, _STAT_SHAPE],
        scratch_shapes=[pltpu.VMEM((1, C), jnp.float32),
                        pltpu.VMEM((1, C), jnp.float32)],
    )(y2, *a, s2, h2, wc, bias)


def _final_call(y3, y2, s2, h2, scale, shift):
    return pl.pallas_call(
        _final_body, grid=(BH, NEB),
        in_specs=[_row2(), _row2(), _stat_spec(), _stat_spec(),
                  _stat_spec(), _stat_spec()],
        out_specs=pl.BlockSpec((1, C, BLK), lambda b, i: (b, 0, i)),
        out_shape=jax.ShapeDtypeStruct((BH, C, E), jnp.float32),
    )(y3, y2, s2, h2, scale, shift)


def kernel(from_up, from_down, gemm_edges, W_up, b_up, W1, b1, W2, b2):
    def wcat(W, cols):
        return jnp.concatenate([W[:, cs, k].T for (cs, k) in cols],
                               axis=0).astype(jnp.bfloat16)

    full = slice(0, C)
    lo, hi = slice(0, C), slice(C, 2 * C)
    taps5 = [0, 1, 2, 3, 4]
    wc1 = wcat(W_up, [(full, k) for k in taps5])
    wcz = wcat(W1, [(hi, k) for k in taps5])
    wc2 = wcat(W1, [(lo, k) for k in taps5])
    wc3 = wcat(W2, [(full, k) for k in taps5])
    bu = b_up.reshape(1, C)
    bz = b1.reshape(1, C)
    b2r = b2.reshape(1, C)

    sc_i = _make_sc_gather(jnp.int32)
    sc_f = _make_sc_gather(jnp.float32)

    H = B // BH
    idxw = []
    for h in range(H):
        sl = slice(h * BH, (h + 1) * BH)
        ge = (gemm_edges[sl].astype(jnp.int32)
              + (jnp.arange(BH, dtype=jnp.int32) * E)[:, None, None])
        idxw.append(ge.reshape(MH, 4).T
                    .reshape(4, NW, NGRP, GCH, KCH)
                    .transpose(1, 2, 3, 0, 4)
                    .reshape(NW, NGRP, GCH * 4, KCH))

    packed = [_pack_call(h, from_up, from_down) for h in range(H)]
    p = [sc_i(packed[h][0], idxw[h]) for h in range(H)]
    t2 = [_conv1_call(packed[h][1], packed[h][2], p[h], wc1, wcz, bu, bz)
          for h in range(H)]
    q = [sc_i(t2[h], idxw[h]) for h in range(H)]
    y2s = [_conv2_call(t2[h], q[h], wc2) for h in range(H)]
    a = [sc_f(y2s[h][0], idxw[h]) for h in range(H)]
    y3s = [_conv3_call(y2s[h][0], a[h], y2s[h][1], y2s[h][2], wc3, b2r)
           for h in range(H)]
    out = [_final_call(y3s[h][0], y2s[h][0], y2s[h][1], y2s[h][2],
                       y3s[h][1], y3s[h][2]) for h in range(H)]
    return jnp.concatenate(out, axis=0)


# speedup vs baseline: 4.0083x; 1.0020x over previous
import jax
import jax.numpy as jnp
import numpy as np
from jax import lax
from jax.experimental import pallas as pl
from jax.experimental.pallas import tpu as pltpu
from jax.experimental.pallas import tpu_sc as plsc

B = 4
E = 80000
M = B * E
C = 128

BH = 2
MH = BH * E

NW = 32
PER_W = MH // NW
KCH = 40
NCHUNK = PER_W // KCH
NGRP = 1
GCH = NCHUNK // NGRP

BLK = 3200
NEB = E // BLK


def _sc_gather_body(table, idxw, o1, o2, o3, o4, ivall,
                    b00, b01, b02, b03, b10, b11, b12, b13,
                    sg0, sg1, sw0, sw1):
    wid = lax.axis_index("s") * 2 + lax.axis_index("c")
    base0 = wid * PER_W

    bufs = ((b00, b01, b02, b03), (b10, b11, b12, b13))
    sgs = (sg0, sg1)
    sws = (sw0, sw1)
    ohs = (o1, o2, o3, o4)

    def group(g, carry):
        pltpu.sync_copy(idxw.at[wid, g], ivall)

        def fire_gathers(lci, s):
            return [pltpu.async_copy(table.at[ivall.at[lci * 4 + j]],
                                     bufs[s][j], sgs[s]) for j in range(4)]

        def fire_writes(lci, s):
            base = base0 + (g * GCH + lci) * KCH
            for j in range(4):
                pltpu.async_copy(bufs[s][j], ohs[j].at[pl.ds(base, KCH)],
                                 sws[s])

        def wait_writes(s):
            for j in range(4):
                pltpu.make_async_copy(bufs[s][j], ohs[j].at[pl.ds(0, KCH)],
                                      sws[s]).wait()

        def run_chunk(lci, s):
            cps = fire_gathers(lci, s)
            for cp in cps:
                cp.wait()
            fire_writes(lci, s)

        run_chunk(0, 0)
        run_chunk(1, 1)

        def pair(k, c2):
            for s in (0, 1):
                wait_writes(s)
                run_chunk(2 + 2 * k + s, s)
            return c2

        lax.fori_loop(0, (GCH - 2) // 2, pair, 0)
        if (GCH - 2) % 2 == 1:
            wait_writes(0)
            run_chunk(GCH - 1, 0)
        wait_writes(0)
        wait_writes(1)
        return carry

    lax.fori_loop(0, NGRP, group, 0)


def _make_sc_gather(dtype):
    mesh = plsc.VectorSubcoreMesh(core_axis_name="c", subcore_axis_name="s")
    out_t = [jax.ShapeDtypeStruct((MH, C), dtype)] * 4
    scratch = ([pltpu.VMEM((GCH * 4, KCH), jnp.int32)]
               + [pltpu.VMEM((KCH, C), dtype)] * 8
               + [pltpu.SemaphoreType.DMA] * 4)
    return pl.kernel(_sc_gather_body, mesh=mesh, out_type=out_t,
                     scratch_types=scratch)


_HI = np.uint32(0xFFFF0000)


def _pack2(a, b):
    ah = lax.bitcast_convert_type(a.astype(jnp.bfloat16).astype(jnp.float32),
                                  jnp.uint32)
    bh = lax.bitcast_convert_type(b.astype(jnp.bfloat16).astype(jnp.float32),
                                  jnp.uint32)
    return lax.bitcast_convert_type((ah & _HI) | (bh >> 16), jnp.int32)


def _unpack_hi(p):
    u = lax.bitcast_convert_type(p, jnp.uint32)
    return lax.bitcast_convert_type(u & _HI, jnp.float32)


def _unpack_lo(p):
    u = lax.bitcast_convert_type(p, jnp.uint32)
    return lax.bitcast_convert_type(u << 16, jnp.float32)


def _combo(a1, a2, a3, a4):
    return [a1 + a3, a2 + a4, jnp.abs(a1 - a3), jnp.abs(a2 - a4)]


def _bf(xs):
    return [x.astype(jnp.bfloat16) for x in xs]


def _pack_body(up, dn, t1, fuo, fdo):
    u = up[...].reshape(C, BLK).T
    d = dn[...].reshape(C, BLK).T
    t1[...] = _pack2(u, d)
    fuo[...] = u
    fdo[...] = d


def _conv1_body(fu, fd, p1, p2, p3, p4, w1, wz, b1, bz, t2):
    hi = [_unpack_hi(p[...]) for p in (p1, p2, p3, p4)]
    lo = [_unpack_lo(p[...]) for p in (p1, p2, p3, p4)]
    gy = jnp.concatenate([fu[...].astype(jnp.bfloat16)] + _bf(_combo(*hi)),
                         axis=1)
    gz = jnp.concatenate([fd[...].astype(jnp.bfloat16)] + _bf(_combo(*lo)),
                         axis=1)
    y1 = jnp.dot(gy, w1[...], preferred_element_type=jnp.float32) + b1[...]
    z = jnp.dot(gz, wz[...], preferred_element_type=jnp.float32) + bz[...]
    t2[...] = _pack2(y1, z)


def _stats_epilogue(i, y, acc1, acc2, scale, shift):
    @pl.when(i == 0)
    def _():
        acc1[...] = jnp.zeros_like(acc1)
        acc2[...] = jnp.zeros_like(acc2)

    acc1[...] += jnp.sum(y, axis=0, keepdims=True)
    acc2[...] += jnp.sum(y * y, axis=0, keepdims=True)

    @pl.when(i == NEB - 1)
    def _():
        mean = acc1[...] * (1.0 / E)
        var = acc2[...] * (1.0 / E) - mean * mean
        rstd = lax.rsqrt(var + 1e-5)
        scale[...] = rstd.reshape(1, 1, C)
        shift[...] = (-mean * rstd).reshape(1, 1, C)


def _conv2_body(t2r, q1, q2, q3, q4, w, y2, scale, shift, acc1, acc2):
    i = pl.program_id(1)
    y1f0 = _unpack_hi(t2r[...])
    z = _unpack_lo(t2r[...])
    qs = [_unpack_hi(q[...]) for q in (q1, q2, q3, q4)]
    g = jnp.concatenate([y1f0.astype(jnp.bfloat16)] + _bf(_combo(*qs)),
                        axis=1)
    y = jnp.dot(g, w[...], preferred_element_type=jnp.float32) + z
    y2[...] = y
    _stats_epilogue(i, y, acc1, acc2, scale, shift)


def _x1(y2val, s2, h2):
    return jnp.maximum(y2val * s2.reshape(1, C) + h2.reshape(1, C), 0.0)


def _conv3_body(y2r, a1, a2, a3, a4, s2, h2, w, bias,
                y3, scale, shift, acc1, acc2):
    i = pl.program_id(1)
    x0 = _x1(y2r[...], s2[...], h2[...])
    xs = [_x1(a[...], s2[...], h2[...]) for a in (a1, a2, a3, a4)]
    g = jnp.concatenate([x0.astype(jnp.bfloat16)] + _bf(_combo(*xs)), axis=1)
    y = jnp.dot(g, w[...], preferred_element_type=jnp.float32) + bias[...]
    y3[...] = y.astype(jnp.bfloat16)
    _stats_epilogue(i, y, acc1, acc2, scale, shift)


def _final_body(y3r, y2r, s2, h2, scale, shift, out):
    r = jnp.maximum(y3r[...] * scale[...].reshape(1, C)
                    + shift[...].reshape(1, C)
                    + _x1(y2r[...], s2[...], h2[...]), 0.0)
    out[...] = r.T.reshape(1, C, BLK)


def _row1():
    return pl.BlockSpec((BLK, C), lambda i: (i, 0))


def _row1_off(h):
    off = h * (MH // BLK)
    return pl.BlockSpec((BLK, C), lambda i, o=off: (i + o, 0))


def _row2():
    return pl.BlockSpec((BLK, C), lambda b, i: (b * NEB + i, 0))


def _w1_spec(k):
    return pl.BlockSpec((k, C), lambda i: (0, 0))


def _w2_spec(k):
    return pl.BlockSpec((k, C), lambda b, i: (0, 0))


def _stat_spec():
    return pl.BlockSpec((1, 1, C), lambda b, i: (b, 0, 0))


_STAT_SHAPE = jax.ShapeDtypeStruct((BH, 1, C), jnp.float32)
_ROW_F32 = jax.ShapeDtypeStruct((MH, C), jnp.float32)
_ROW_BF16 = jax.ShapeDtypeStruct((MH, C), jnp.bfloat16)


def _pack_call(h, from_up, from_down):
    spec = pl.BlockSpec((1, C, BLK),
                        lambda i, hh=h: (hh * BH + i // NEB, 0, i % NEB))
    return pl.pallas_call(
        _pack_body, grid=(MH // BLK,),
        in_specs=[spec, spec],
        out_specs=[_row1(), _row1(), _row1()],
        out_shape=[jax.ShapeDtypeStruct((MH, C), jnp.int32),
                   _ROW_F32, _ROW_F32],
    )(from_up, from_down)


def _conv1_call(fu, fd, p, w1, wz, b1, bz):
    return pl.pallas_call(
        _conv1_body, grid=(MH // BLK,),
        in_specs=[_row1()] * 6
                 + [_w1_spec(5 * C), _w1_spec(5 * C),
                    _w1_spec(1), _w1_spec(1)],
        out_specs=_row1(),
        out_shape=jax.ShapeDtypeStruct((MH, C), jnp.int32),
    )(fu, fd, *p, w1, wz, b1, bz)


def _conv2_call(t2, q, wc):
    return pl.pallas_call(
        _conv2_body, grid=(BH, NEB),
        in_specs=[_row2()] * 5 + [_w2_spec(5 * C)],
        out_specs=[_row2(), _stat_spec(), _stat_spec()],
        out_shape=[_ROW_F32, _STAT_SHAPE, _STAT_SHAPE],
        scratch_shapes=[pltpu.VMEM((1, C), jnp.float32),
                        pltpu.VMEM((1, C), jnp.float32)],
    )(t2, *q, wc)


def _conv3_call(y2, a, s2, h2, wc, bias):
    return pl.pallas_call(
        _conv3_body, grid=(BH, NEB),
        in_specs=([_row2()] * 5 + [_stat_spec(), _stat_spec()]
                  + [_w2_spec(5 * C), _w2_spec(1)]),
        out_specs=[_row2(), _stat_spec(), _stat_spec()],
        out_shape=[_ROW_BF16, _STAT_SHAPE, _STAT_SHAPE],
        scratch_shapes=[pltpu.VMEM((1, C), jnp.float32),
                        pltpu.VMEM((1, C), jnp.float32)],
    )(y2, *a, s2, h2, wc, bias)


def _final_call(y3, y2, s2, h2, scale, shift):
    return pl.pallas_call(
        _final_body, grid=(BH, NEB),
        in_specs=[_row2(), _row2(), _stat_spec(), _stat_spec(),
                  _stat_spec(), _stat_spec()],
        out_specs=pl.BlockSpec((1, C, BLK), lambda b, i: (b, 0, i)),
        out_shape=jax.ShapeDtypeStruct((BH, C, E), jnp.float32),
    )(y3, y2, s2, h2, scale, shift)


def kernel(from_up, from_down, gemm_edges, W_up, b_up, W1, b1, W2, b2):
    def wcat(W, cols):
        return jnp.concatenate([W[:, cs, k].T for (cs, k) in cols],
                               axis=0).astype(jnp.bfloat16)

    full = slice(0, C)
    lo, hi = slice(0, C), slice(C, 2 * C)
    taps5 = [0, 1, 2, 3, 4]
    wc1 = wcat(W_up, [(full, k) for k in taps5])
    wcz = wcat(W1, [(hi, k) for k in taps5])
    wc2 = wcat(W1, [(lo, k) for k in taps5])
    wc3 = wcat(W2, [(full, k) for k in taps5])
    bu = b_up.reshape(1, C)
    bz = b1.reshape(1, C)
    b2r = b2.reshape(1, C)

    sc_i = _make_sc_gather(jnp.int32)
    sc_f = _make_sc_gather(jnp.float32)

    H = B // BH
    idxw = []
    for h in range(H):
        sl = slice(h * BH, (h + 1) * BH)
        ge = (gemm_edges[sl].astype(jnp.int32)
              + (jnp.arange(BH, dtype=jnp.int32) * E)[:, None, None])
        idxw.append(ge.reshape(MH, 4).T
                    .reshape(4, NW, NGRP, GCH, KCH)
                    .transpose(1, 2, 3, 0, 4)
                    .reshape(NW, NGRP, GCH * 4, KCH))

    packed = [_pack_call(h, from_up, from_down) for h in range(H)]
    p = [sc_i(packed[h][0], idxw[h]) for h in range(H)]
    t2 = [_conv1_call(packed[h][1], packed[h][2], p[h], wc1, wcz, bu, bz)
          for h in range(H)]
    q = [sc_i(t2[h], idxw[h]) for h in range(H)]
    y2s = [_conv2_call(t2[h], q[h], wc2) for h in range(H)]
    a = [sc_f(y2s[h][0], idxw[h]) for h in range(H)]
    y3s = [_conv3_call(y2s[h][0], a[h], y2s[h][1], y2s[h][2], wc3, b2r)
           for h in range(H)]
    out = [_final_call(y3s[h][0], y2s[h][0], y2s[h][1], y2s[h][2],
                       y3s[h][1], y3s[h][2]) for h in range(H)]
    return jnp.concatenate(out, axis=0)
